# Initial kernel scaffold; baseline (speedup 1.0000x reference)
import jax
import jax.numpy as jnp
from jax.experimental import pallas as pl


def kernel(x, edge_index, W1_0, W1_1, W1_2, b1_0, b1_1, b1_2, W2_0, W2_1, W2_2, b2_0, b2_1, b2_2, fc_W, fc_b):
    raise NotImplementedError("write your pallas kernel here")


# speedup vs baseline: 17.2994x; 17.2994x over previous
import jax
import jax.numpy as jnp
from jax import lax
from jax.experimental import pallas as pl
from jax.experimental.pallas import tpu as pltpu
from jax.experimental.pallas import tpu_sc as plsc

NC = 2
NS = 16
NW = NC * NS
LANES = 16
BR = 1000


def _make_sc_scatter(n_rows, pad_rows, W, nchunk, ch, n_z):
  NZ = 128
  acc_rows = -(-(n_rows + pad_rows) // NZ) * NZ
  nzch = acc_rows // NZ
  wb = next(c for c in range(128, 0, -8) if n_rows % c == 0)
  nwb = n_rows // wb
  npass = max(n_z, 1)

  mesh = plsc.VectorSubcoreMesh(
      core_axis_name="c", subcore_axis_name="s",
      num_cores=NC, num_subcores=NS)

  scratch = [
      pltpu.VMEM((nchunk, ch), jnp.int32),
      pltpu.VMEM((nchunk, ch), jnp.int32),
      pltpu.VMEM((ch, W), jnp.float32),
      pltpu.VMEM((ch, W), jnp.float32),
      pltpu.VMEM((NZ, W), jnp.float32),
      pltpu.VMEM_SHARED((acc_rows, W), jnp.float32),
      pltpu.SemaphoreType.DMA,
      pltpu.SemaphoreType.DMA,
  ]

  def body(*refs):
    z_refs = refs[:n_z]
    src_hbm, dst_hbm, out_hbm = refs[n_z:n_z + 3]
    src_v, dst_v, bufa, bufb, zbuf, acc, sema, semb = refs[n_z + 3:]
    c = lax.axis_index("c")
    s = lax.axis_index("s")
    wid = c * NS + s

    @pl.loop(0, NZ)
    def _(i):
      @pl.loop(0, W // LANES)
      def _(k):
        zbuf[i, pl.ds(k * LANES, LANES)] = jnp.zeros((LANES,), jnp.float32)

    pltpu.sync_copy(dst_hbm.at[wid], dst_v)
    if n_z:
      pltpu.sync_copy(src_hbm.at[wid], src_v)
    else:
      @pl.loop(0, ch)
      def _(i):
        @pl.loop(0, W // LANES)
        def _(k):
          bufa[i, pl.ds(k * LANES, LANES)] = jnp.ones((LANES,), jnp.float32)

    for p in range(npass):
      @pl.loop(0, -(-nzch // NS))
      def _(k):
        idx = k * NS + s

        @pl.when(idx < nzch)
        def _():
          pltpu.sync_copy(zbuf, acc.at[pl.ds(idx * NZ, NZ)])

      plsc.subcore_barrier()

      if n_z:
        z_hbm = z_refs[p]
        pltpu.async_copy(z_hbm.at[src_v.at[0]], bufa, sema)

        @pl.loop(0, nchunk // 2)
        def _(j):
          pltpu.make_async_copy(z_hbm.at[src_v.at[2 * j]], bufa, sema).wait()
          pltpu.async_copy(z_hbm.at[src_v.at[2 * j + 1]], bufb, semb)
          pltpu.sync_copy(bufa, acc.at[dst_v.at[2 * j]], add=True)
          pltpu.make_async_copy(
              z_hbm.at[src_v.at[2 * j + 1]], bufb, semb).wait()

          @pl.when(2 * j + 2 < nchunk)
          def _():
            pltpu.async_copy(z_hbm.at[src_v.at[2 * j + 2]], bufa, sema)

          pltpu.sync_copy(bufb, acc.at[dst_v.at[2 * j + 1]], add=True)

        if nchunk % 2 == 1:
          pltpu.make_async_copy(
              z_hbm.at[src_v.at[nchunk - 1]], bufa, sema).wait()
          pltpu.sync_copy(bufa, acc.at[dst_v.at[nchunk - 1]], add=True)
      else:
        @pl.loop(0, nchunk)
        def _(j):
          pltpu.sync_copy(bufa, acc.at[dst_v.at[j]], add=True)

      plsc.subcore_barrier()

      @pl.loop(0, -(-nwb // NS))
      def _(k):
        idx = k * NS + s

        @pl.when(idx < nwb)
        def _():
          pltpu.sync_copy(acc.at[pl.ds(idx * wb, wb)],
                          out_hbm.at[c, p, pl.ds(idx * wb, wb)])

      if p + 1 < npass:
        plsc.subcore_barrier()

  return pl.kernel(
      body,
      out_type=jax.ShapeDtypeStruct((NC, npass, n_rows, W), jnp.float32),
      mesh=mesh,
      scratch_types=scratch,
      compiler_params=pltpu.CompilerParams(use_tc_tiling_on_sc=False),
  )


def _rows(w):
  return pl.BlockSpec((BR, w), lambda i: (i, 0))


def _prows(p, w):
  return pl.BlockSpec((NC, p, BR, w), lambda i: (0, 0, i, 0))


def _full2(a):
  return pl.BlockSpec(a.shape, lambda i: (0, 0))


def _tc_call(body, n, out_ws, in_arrays, in_specs):
  return pl.pallas_call(
      body,
      grid=(n // BR,),
      in_specs=in_specs,
      out_specs=tuple(_rows(w) for w in out_ws),
      out_shape=tuple(
          jax.ShapeDtypeStruct((n, w), jnp.float32) for w in out_ws),
  )(*in_arrays)


def _dot(a, b):
  return jnp.dot(a, b, preferred_element_type=jnp.float32)


def _tc1_body(dp, x, dinv_o, z1a_o, z1b_o):
  deg = dp[0, 0, :, :1] + dp[1, 0, :, :1] + 1.0
  dinv = lax.rsqrt(deg)
  dinv_o[...] = dinv
  z1a_o[...] = dinv * x[:, :64]
  z1b_o[...] = dinv * x[:, 64:]


def _tc2_body(sp, z1a, z1b, dinv, u1_o, z2a_o, z2b_o):
  dv = dinv[...]
  ua = dv * (sp[0, 0] + sp[1, 0] + z1a[...])
  ub = dv * (sp[0, 1] + sp[1, 1] + z1b[...])
  u1_o[...] = jnp.concatenate([ua, ub], axis=1)
  z2a_o[...] = dv * ua
  z2b_o[...] = dv * ub


def _tc3_body(sp, z2a, z2b, dinv, x, u1, W10, b10, W11, b11, W12, b12,
              W20, b20, W21, W22, q0_o, ga_o, gb_o):
  dv = dinv[...]
  u2 = dv * jnp.concatenate(
      [sp[0, 0] + sp[1, 0] + z2a[...], sp[0, 1] + sp[1, 1] + z2b[...]],
      axis=1)
  h = jnp.concatenate([
      _dot(x[...], W10[...]) + b10[...],
      _dot(u1[...], W11[...]) + b11[...],
      _dot(u2, W12[...]) + b12[...],
  ], axis=1)
  h = jnp.maximum(h, 0.0)
  q0_o[...] = _dot(h, W20[...]) + b20[...]
  zpad = jnp.zeros((h.shape[0], 4), jnp.float32)
  ga_o[...] = dv * jnp.concatenate([_dot(h, W21[...]), zpad], axis=1)
  gb_o[...] = dv * jnp.concatenate([_dot(h, W22[...]), zpad], axis=1)


def _tc4_body(sp, ga, gb, dinv, v1_o, z4_o):
  dv = dinv[...]
  v1_o[...] = dv * (sp[0, 0] + sp[1, 0] + ga[...])
  z4_o[...] = dv * dv * (sp[0, 1] + sp[1, 1] + gb[...])


def _tc5_body(sp, z4, dinv, q0, v1, b21, b22, fcW, fcb, out_o):
  dv = dinv[...]
  w = dv * (sp[0, 0] + sp[1, 0] + z4[...])
  h2 = jnp.concatenate([
      q0[...],
      v1[:, :60] + b21[...],
      w[:, :60] + b22[...],
  ], axis=1)
  h2 = jnp.maximum(h2, 0.0)
  out_o[...] = _dot(h2, fcW[...]) + fcb[...]


def kernel(x, edge_index, W1_0, W1_1, W1_2, b1_0, b1_1, b1_2,
           W2_0, W2_1, W2_2, b2_0, b2_1, b2_2, fc_W, fc_b):
  n, d_in = x.shape
  e = edge_index.shape[1]
  src = edge_index[0].astype(jnp.int32)
  dst = edge_index[1].astype(jnp.int32)

  ch = None
  for c in range(128, 0, -8):
    if e % (NW * c) == 0:
      ch = c
      break
  if ch is not None:
    pad_rows = 0
    ep = e
  else:
    ch = 64
    ep = -(-e // (NW * ch)) * (NW * ch)
    pad_rows = 16
    fill = jnp.full((ep - e,), n, jnp.int32)
    src = jnp.concatenate([src, fill])
    dst = jnp.concatenate([dst, fill])
  nchunk = ep // (NW * ch)
  src_r = src.reshape(NW, nchunk, ch)
  dst_r = dst.reshape(NW, nchunk, ch)

  def maybe_pad(z):
    if pad_rows:
      return jnp.concatenate(
          [z, jnp.zeros((pad_rows, z.shape[1]), z.dtype)], axis=0)
    return z

  deg_k = _make_sc_scatter(n, pad_rows, 16, nchunk, ch, n_z=0)
  sc2 = _make_sc_scatter(n, pad_rows, 64, nchunk, ch, n_z=2)
  sc1 = _make_sc_scatter(n, pad_rows, 64, nchunk, ch, n_z=1)

  degp = deg_k(src_r, dst_r)

  b10 = b1_0.reshape(1, -1)
  b11 = b1_1.reshape(1, -1)
  b12 = b1_2.reshape(1, -1)
  b20 = b2_0.reshape(1, -1)
  b21 = b2_1.reshape(1, -1)
  b22 = b2_2.reshape(1, -1)
  fcb = fc_b.reshape(1, -1)

  dinv, z1a, z1b = _tc_call(
      _tc1_body, n, (1, 64, 64),
      (degp, x), (_prows(1, 16), _rows(d_in)))

  sp = sc2(maybe_pad(z1a), maybe_pad(z1b), src_r, dst_r)
  u1, z2a, z2b = _tc_call(
      _tc2_body, n, (d_in, 64, 64),
      (sp, z1a, z1b, dinv),
      (_prows(2, 64), _rows(64), _rows(64), _rows(1)))

  sp = sc2(maybe_pad(z2a), maybe_pad(z2b), src_r, dst_r)
  q0, ga, gb = _tc_call(
      _tc3_body, n, (60, 64, 64),
      (sp, z2a, z2b, dinv, x, u1,
       W1_0, b10, W1_1, b11, W1_2, b12, W2_0, b20, W2_1, W2_2),
      (_prows(2, 64), _rows(64), _rows(64), _rows(1), _rows(d_in),
       _rows(d_in),
       _full2(W1_0), _full2(b10), _full2(W1_1), _full2(b11),
       _full2(W1_2), _full2(b12), _full2(W2_0), _full2(b20),
       _full2(W2_1), _full2(W2_2)))

  sp = sc2(maybe_pad(ga), maybe_pad(gb), src_r, dst_r)
  v1, z4 = _tc_call(
      _tc4_body, n, (64, 64),
      (sp, ga, gb, dinv),
      (_prows(2, 64), _rows(64), _rows(64), _rows(1)))

  sp = sc1(maybe_pad(z4), src_r, dst_r)
  (out,) = _tc_call(
      _tc5_body, n, (128,),
      (sp, z4, dinv, q0, v1, b21, b22, fc_W, fcb),
      (_prows(1, 64), _rows(64), _rows(1), _rows(60), _rows(64),
       _full2(b21), _full2(b22), _full2(fc_W), _full2(fcb)))
  return out


# speedup vs baseline: 20.8557x; 1.2056x over previous
import jax
import jax.numpy as jnp
from jax import lax
from jax.experimental import pallas as pl
from jax.experimental.pallas import tpu as pltpu
from jax.experimental.pallas import tpu_sc as plsc

NC = 2
NS = 16
NW = NC * NS
LANES = 16
BR = 1000


def _make_sc_scatter(n_rows, pad_rows, W, nchunk, ch, n_z):
  NZ = 128
  acc_rows = -(-(n_rows + pad_rows) // NZ) * NZ
  nzch = acc_rows // NZ
  wb = next(c for c in range(128, 0, -8) if n_rows % c == 0)
  nwb = n_rows // wb
  npass = max(n_z, 1)

  mesh = plsc.VectorSubcoreMesh(
      core_axis_name="c", subcore_axis_name="s",
      num_cores=NC, num_subcores=NS)

  scratch = [
      pltpu.VMEM((nchunk, ch), jnp.int32),
      pltpu.VMEM((nchunk, ch), jnp.int32),
      pltpu.VMEM((ch, W), jnp.float32),
      pltpu.VMEM((ch, W), jnp.float32),
      pltpu.VMEM((NZ, W), jnp.float32),
      pltpu.VMEM_SHARED((acc_rows, W), jnp.float32),
      pltpu.SemaphoreType.DMA,
      pltpu.SemaphoreType.DMA,
  ]

  def body(*refs):
    z_refs = refs[:n_z]
    src_hbm, dst_hbm, out_hbm = refs[n_z:n_z + 3]
    src_v, dst_v, bufa, bufb, zbuf, acc, sema, semb = refs[n_z + 3:]
    c = lax.axis_index("c")
    s = lax.axis_index("s")
    wid = c * NS + s

    @pl.loop(0, NZ)
    def _(i):
      @pl.loop(0, W // LANES)
      def _(k):
        zbuf[i, pl.ds(k * LANES, LANES)] = jnp.zeros((LANES,), jnp.float32)

    pltpu.sync_copy(dst_hbm.at[wid], dst_v)
    if n_z:
      pltpu.sync_copy(src_hbm.at[wid], src_v)
    else:
      @pl.loop(0, ch)
      def _(i):
        @pl.loop(0, W // LANES)
        def _(k):
          bufa[i, pl.ds(k * LANES, LANES)] = jnp.ones((LANES,), jnp.float32)

    for p in range(npass):
      @pl.loop(0, -(-nzch // NS))
      def _(k):
        idx = k * NS + s

        @pl.when(idx < nzch)
        def _():
          pltpu.sync_copy(zbuf, acc.at[pl.ds(idx * NZ, NZ)])

      plsc.subcore_barrier()

      if n_z:
        z_hbm = z_refs[p]
        pltpu.async_copy(z_hbm.at[src_v.at[0]], bufa, sema)

        @pl.loop(0, nchunk // 2)
        def _(j):
          pltpu.make_async_copy(z_hbm.at[src_v.at[2 * j]], bufa, sema).wait()
          pltpu.async_copy(z_hbm.at[src_v.at[2 * j + 1]], bufb, semb)
          pltpu.sync_copy(bufa, acc.at[dst_v.at[2 * j]], add=True)
          pltpu.make_async_copy(
              z_hbm.at[src_v.at[2 * j + 1]], bufb, semb).wait()

          @pl.when(2 * j + 2 < nchunk)
          def _():
            pltpu.async_copy(z_hbm.at[src_v.at[2 * j + 2]], bufa, sema)

          pltpu.sync_copy(bufb, acc.at[dst_v.at[2 * j + 1]], add=True)

        if nchunk % 2 == 1:
          pltpu.make_async_copy(
              z_hbm.at[src_v.at[nchunk - 1]], bufa, sema).wait()
          pltpu.sync_copy(bufa, acc.at[dst_v.at[nchunk - 1]], add=True)
      else:
        @pl.loop(0, nchunk)
        def _(j):
          pltpu.sync_copy(bufa, acc.at[dst_v.at[j]], add=True)

      plsc.subcore_barrier()

      @pl.loop(0, -(-nwb // NS))
      def _(k):
        idx = k * NS + s

        @pl.when(idx < nwb)
        def _():
          pltpu.sync_copy(acc.at[pl.ds(idx * wb, wb)],
                          out_hbm.at[c, p, pl.ds(idx * wb, wb)])

      if p + 1 < npass:
        plsc.subcore_barrier()

  return pl.kernel(
      body,
      out_type=jax.ShapeDtypeStruct((NC, npass, n_rows, W), jnp.float32),
      mesh=mesh,
      scratch_types=scratch,
      compiler_params=pltpu.CompilerParams(use_tc_tiling_on_sc=False),
  )


def _rows(w):
  return pl.BlockSpec((BR, w), lambda i: (i, 0))


def _prows(p, w):
  return pl.BlockSpec((NC, p, BR, w), lambda i: (0, 0, i, 0))


def _full2(a):
  return pl.BlockSpec(a.shape, lambda i: (0, 0))


def _tc_call(body, n, out_ws, in_arrays, in_specs):
  return pl.pallas_call(
      body,
      grid=(n // BR,),
      in_specs=in_specs,
      out_specs=tuple(_rows(w) for w in out_ws),
      out_shape=tuple(
          jax.ShapeDtypeStruct((n, w), jnp.float32) for w in out_ws),
  )(*in_arrays)


def _dot(a, b):
  return jnp.dot(a, b, preferred_element_type=jnp.float32)


def _tc1_body(dp, x, dinv_o, z1a_o, z1b_o):
  deg = dp[0, 0, :, :1] + dp[1, 0, :, :1] + 1.0
  dinv = lax.rsqrt(deg)
  dinv_o[...] = dinv
  z1a_o[...] = dinv * x[:, :64]
  z1b_o[...] = dinv * x[:, 64:]


def _tc2_body(sp, z1a, z1b, dinv, u1_o, z2a_o, z2b_o):
  dv = dinv[...]
  ua = dv * (sp[0, 0] + sp[1, 0] + z1a[...])
  ub = dv * (sp[0, 1] + sp[1, 1] + z1b[...])
  u1_o[...] = jnp.concatenate([ua, ub], axis=1)
  z2a_o[...] = dv * ua
  z2b_o[...] = dv * ub


def _tc3_body(sp, z2a, z2b, dinv, x, u1, W10, b10, W11, b11, W12, b12,
              W20, b20, W21, W22, q0_o, ga_o, gb_o):
  dv = dinv[...]
  u2 = dv * jnp.concatenate(
      [sp[0, 0] + sp[1, 0] + z2a[...], sp[0, 1] + sp[1, 1] + z2b[...]],
      axis=1)
  h = jnp.concatenate([
      _dot(x[...], W10[...]) + b10[...],
      _dot(u1[...], W11[...]) + b11[...],
      _dot(u2, W12[...]) + b12[...],
  ], axis=1)
  h = jnp.maximum(h, 0.0)
  q0_o[...] = _dot(h, W20[...]) + b20[...]
  zpad = jnp.zeros((h.shape[0], 4), jnp.float32)
  ga_o[...] = dv * jnp.concatenate([_dot(h, W21[...]), zpad], axis=1)
  gb_o[...] = dv * jnp.concatenate([_dot(h, W22[...]), zpad], axis=1)


def _tc4_body(sp, ga, gb, dinv, v1_o, z4_o):
  dv = dinv[...]
  v1_o[...] = dv * (sp[0, 0] + sp[1, 0] + ga[...])
  z4_o[...] = dv * dv * (sp[0, 1] + sp[1, 1] + gb[...])


def _tc5_body(sp, z4, dinv, q0, v1, b21, b22, fcW, fcb, out_o):
  dv = dinv[...]
  w = dv * (sp[0, 0] + sp[1, 0] + z4[...])
  h2 = jnp.concatenate([
      q0[...],
      v1[:, :60] + b21[...],
      w[:, :60] + b22[...],
  ], axis=1)
  h2 = jnp.maximum(h2, 0.0)
  out_o[...] = _dot(h2, fcW[...]) + fcb[...]


def kernel(x, edge_index, W1_0, W1_1, W1_2, b1_0, b1_1, b1_2,
           W2_0, W2_1, W2_2, b2_0, b2_1, b2_2, fc_W, fc_b):
  n, d_in = x.shape
  e = edge_index.shape[1]
  src = edge_index[0].astype(jnp.int32)
  dst = edge_index[1].astype(jnp.int32)

  ch = None
  for c in range(128, 0, -1):
    if e % (NW * c) == 0:
      ch = c
      break
  if ch is not None:
    pad_rows = 0
    ep = e
  else:
    ch = 64
    ep = -(-e // (NW * ch)) * (NW * ch)
    pad_rows = 16
    fill = jnp.full((ep - e,), n, jnp.int32)
    src = jnp.concatenate([src, fill])
    dst = jnp.concatenate([dst, fill])
  nchunk = ep // (NW * ch)
  src_r = src.reshape(NW, nchunk, ch)
  dst_r = dst.reshape(NW, nchunk, ch)

  def maybe_pad(z):
    if pad_rows:
      return jnp.concatenate(
          [z, jnp.zeros((pad_rows, z.shape[1]), z.dtype)], axis=0)
    return z

  deg_k = _make_sc_scatter(n, pad_rows, 16, nchunk, ch, n_z=0)
  sc2 = _make_sc_scatter(n, pad_rows, 64, nchunk, ch, n_z=2)
  sc1 = _make_sc_scatter(n, pad_rows, 64, nchunk, ch, n_z=1)

  degp = deg_k(src_r, dst_r)

  b10 = b1_0.reshape(1, -1)
  b11 = b1_1.reshape(1, -1)
  b12 = b1_2.reshape(1, -1)
  b20 = b2_0.reshape(1, -1)
  b21 = b2_1.reshape(1, -1)
  b22 = b2_2.reshape(1, -1)
  fcb = fc_b.reshape(1, -1)

  dinv, z1a, z1b = _tc_call(
      _tc1_body, n, (1, 64, 64),
      (degp, x), (_prows(1, 16), _rows(d_in)))

  sp = sc2(maybe_pad(z1a), maybe_pad(z1b), src_r, dst_r)
  u1, z2a, z2b = _tc_call(
      _tc2_body, n, (d_in, 64, 64),
      (sp, z1a, z1b, dinv),
      (_prows(2, 64), _rows(64), _rows(64), _rows(1)))

  sp = sc2(maybe_pad(z2a), maybe_pad(z2b), src_r, dst_r)
  q0, ga, gb = _tc_call(
      _tc3_body, n, (60, 64, 64),
      (sp, z2a, z2b, dinv, x, u1,
       W1_0, b10, W1_1, b11, W1_2, b12, W2_0, b20, W2_1, W2_2),
      (_prows(2, 64), _rows(64), _rows(64), _rows(1), _rows(d_in),
       _rows(d_in),
       _full2(W1_0), _full2(b10), _full2(W1_1), _full2(b11),
       _full2(W1_2), _full2(b12), _full2(W2_0), _full2(b20),
       _full2(W2_1), _full2(W2_2)))

  sp = sc2(maybe_pad(ga), maybe_pad(gb), src_r, dst_r)
  v1, z4 = _tc_call(
      _tc4_body, n, (64, 64),
      (sp, ga, gb, dinv),
      (_prows(2, 64), _rows(64), _rows(64), _rows(1)))

  sp = sc1(maybe_pad(z4), src_r, dst_r)
  (out,) = _tc_call(
      _tc5_body, n, (128,),
      (sp, z4, dinv, q0, v1, b21, b22, fc_W, fcb),
      (_prows(1, 64), _rows(64), _rows(1), _rows(60), _rows(64),
       _full2(b21), _full2(b22), _full2(fc_W), _full2(fcb)))
  return out


# speedup vs baseline: 27.3163x; 1.3098x over previous
import jax
import jax.numpy as jnp
from jax import lax
from jax.experimental import pallas as pl
from jax.experimental.pallas import tpu as pltpu
from jax.experimental.pallas import tpu_sc as plsc

NC = 2
NS = 16
NW = NC * NS
LANES = 16
BR = 1000


def _make_sc_scatter(n_rows, pad_rows, W, nchunk, ch, n_z):
  NZ = 128
  acc_rows = -(-(n_rows + pad_rows) // NZ) * NZ
  nzch = acc_rows // NZ
  wb = next(c for c in range(128, 0, -8) if n_rows % c == 0)
  nwb = n_rows // wb
  npass = max(n_z, 1)

  mesh = plsc.VectorSubcoreMesh(
      core_axis_name="c", subcore_axis_name="s",
      num_cores=NC, num_subcores=NS)

  NB = 4 if (n_z and nchunk % 4 == 0 and nchunk >= 8) else 2
  scratch = [
      pltpu.VMEM((nchunk, ch), jnp.int32),
      pltpu.VMEM((nchunk, ch), jnp.int32),
  ] + [pltpu.VMEM((ch, W), jnp.float32) for _ in range(NB)] + [
      pltpu.VMEM((NZ, W), jnp.float32),
      pltpu.VMEM_SHARED((acc_rows, W), jnp.float32),
  ] + [pltpu.SemaphoreType.DMA for _ in range(2 * NB)]

  def body(*refs):
    z_refs = refs[:n_z]
    src_hbm, dst_hbm, out_hbm = refs[n_z:n_z + 3]
    src_v, dst_v = refs[n_z + 3:n_z + 5]
    bufs = refs[n_z + 5:n_z + 5 + NB]
    zbuf = refs[n_z + 5 + NB]
    acc = refs[n_z + 6 + NB]
    gsems = refs[n_z + 7 + NB:n_z + 7 + 2 * NB]
    ssems = refs[n_z + 7 + 2 * NB:]
    bufa, sema = bufs[0], gsems[0]
    c = lax.axis_index("c")
    s = lax.axis_index("s")
    wid = c * NS + s

    @pl.loop(0, NZ)
    def _(i):
      @pl.loop(0, W // LANES)
      def _(k):
        zbuf[i, pl.ds(k * LANES, LANES)] = jnp.zeros((LANES,), jnp.float32)

    pltpu.sync_copy(dst_hbm.at[wid], dst_v)
    if n_z:
      pltpu.sync_copy(src_hbm.at[wid], src_v)
    else:
      @pl.loop(0, ch)
      def _(i):
        @pl.loop(0, W // LANES)
        def _(k):
          bufa[i, pl.ds(k * LANES, LANES)] = jnp.ones((LANES,), jnp.float32)

    for p in range(npass):
      @pl.loop(0, -(-nzch // NS))
      def _(k):
        idx = k * NS + s

        @pl.when(idx < nzch)
        def _():
          pltpu.sync_copy(zbuf, acc.at[pl.ds(idx * NZ, NZ)])

      plsc.subcore_barrier()

      if n_z:
        z_hbm = z_refs[p]
        if NB == 4:
          for k in range(4):
            pltpu.async_copy(z_hbm.at[src_v.at[k]], bufs[k], gsems[k])

          @pl.loop(0, nchunk // 4)
          def _(j):
            base = 4 * j
            for k in range(4):
              pltpu.make_async_copy(
                  z_hbm.at[src_v.at[base + k]], bufs[k], gsems[k]).wait()
              pltpu.async_copy(
                  bufs[k], acc.at[dst_v.at[base + k]], ssems[k], add=True)
            for k in range(4):
              pltpu.make_async_copy(
                  bufs[k], acc.at[dst_v.at[base + k]], ssems[k]).wait()

              @pl.when(base + 4 + k < nchunk)
              def _(k=k):
                pltpu.async_copy(
                    z_hbm.at[src_v.at[base + 4 + k]], bufs[k], gsems[k])
        else:
          bufb, semb = bufs[1], gsems[1]
          pltpu.async_copy(z_hbm.at[src_v.at[0]], bufa, sema)

          @pl.loop(0, nchunk // 2)
          def _(j):
            pltpu.make_async_copy(
                z_hbm.at[src_v.at[2 * j]], bufa, sema).wait()
            pltpu.async_copy(z_hbm.at[src_v.at[2 * j + 1]], bufb, semb)
            pltpu.sync_copy(bufa, acc.at[dst_v.at[2 * j]], add=True)
            pltpu.make_async_copy(
                z_hbm.at[src_v.at[2 * j + 1]], bufb, semb).wait()

            @pl.when(2 * j + 2 < nchunk)
            def _():
              pltpu.async_copy(z_hbm.at[src_v.at[2 * j + 2]], bufa, sema)

            pltpu.sync_copy(bufb, acc.at[dst_v.at[2 * j + 1]], add=True)

          if nchunk % 2 == 1:
            pltpu.make_async_copy(
                z_hbm.at[src_v.at[nchunk - 1]], bufa, sema).wait()
            pltpu.sync_copy(bufa, acc.at[dst_v.at[nchunk - 1]], add=True)
      else:
        @pl.loop(0, nchunk)
        def _(j):
          pltpu.sync_copy(bufa, acc.at[dst_v.at[j]], add=True)

      plsc.subcore_barrier()

      @pl.loop(0, -(-nwb // NS))
      def _(k):
        idx = k * NS + s

        @pl.when(idx < nwb)
        def _():
          pltpu.sync_copy(acc.at[pl.ds(idx * wb, wb)],
                          out_hbm.at[c, p, pl.ds(idx * wb, wb)])

      if p + 1 < npass:
        plsc.subcore_barrier()

  return pl.kernel(
      body,
      out_type=jax.ShapeDtypeStruct((NC, npass, n_rows, W), jnp.float32),
      mesh=mesh,
      scratch_types=scratch,
      compiler_params=pltpu.CompilerParams(use_tc_tiling_on_sc=False),
  )


def _rows(w):
  return pl.BlockSpec((BR, w), lambda i: (i, 0))


def _prows(p, w):
  return pl.BlockSpec((NC, p, BR, w), lambda i: (0, 0, i, 0))


def _full2(a):
  return pl.BlockSpec(a.shape, lambda i: (0, 0))


def _tc_call(body, n, out_ws, in_arrays, in_specs):
  return pl.pallas_call(
      body,
      grid=(n // BR,),
      in_specs=in_specs,
      out_specs=tuple(_rows(w) for w in out_ws),
      out_shape=tuple(
          jax.ShapeDtypeStruct((n, w), jnp.float32) for w in out_ws),
  )(*in_arrays)


def _dot(a, b):
  return jnp.dot(a, b, preferred_element_type=jnp.float32)


def _tc1_body(dp, x, dinv_o, z1a_o, z1b_o):
  deg = dp[0, 0, :, :1] + dp[1, 0, :, :1] + 1.0
  dinv = lax.rsqrt(deg)
  dinv_o[...] = dinv
  z1a_o[...] = dinv * x[:, :64]
  z1b_o[...] = dinv * x[:, 64:]


def _tc2_body(sp, z1a, z1b, dinv, u1_o, z2a_o, z2b_o):
  dv = dinv[...]
  ua = dv * (sp[0, 0] + sp[1, 0] + z1a[...])
  ub = dv * (sp[0, 1] + sp[1, 1] + z1b[...])
  u1_o[...] = jnp.concatenate([ua, ub], axis=1)
  z2a_o[...] = dv * ua
  z2b_o[...] = dv * ub


def _tc3_body(sp, z2a, z2b, dinv, x, u1, W10, b10, W11, b11, W12, b12,
              W20, b20, W21, W22, q0_o, ga_o, gb_o):
  dv = dinv[...]
  u2 = dv * jnp.concatenate(
      [sp[0, 0] + sp[1, 0] + z2a[...], sp[0, 1] + sp[1, 1] + z2b[...]],
      axis=1)
  h = jnp.concatenate([
      _dot(x[...], W10[...]) + b10[...],
      _dot(u1[...], W11[...]) + b11[...],
      _dot(u2, W12[...]) + b12[...],
  ], axis=1)
  h = jnp.maximum(h, 0.0)
  q0_o[...] = _dot(h, W20[...]) + b20[...]
  zpad = jnp.zeros((h.shape[0], 4), jnp.float32)
  ga_o[...] = dv * jnp.concatenate([_dot(h, W21[...]), zpad], axis=1)
  gb_o[...] = dv * jnp.concatenate([_dot(h, W22[...]), zpad], axis=1)


def _tc4_body(sp, ga, gb, dinv, v1_o, z4_o):
  dv = dinv[...]
  v1_o[...] = dv * (sp[0, 0] + sp[1, 0] + ga[...])
  z4_o[...] = dv * dv * (sp[0, 1] + sp[1, 1] + gb[...])


def _tc5_body(sp, z4, dinv, q0, v1, b21, b22, fcW, fcb, out_o):
  dv = dinv[...]
  w = dv * (sp[0, 0] + sp[1, 0] + z4[...])
  h2 = jnp.concatenate([
      q0[...],
      v1[:, :60] + b21[...],
      w[:, :60] + b22[...],
  ], axis=1)
  h2 = jnp.maximum(h2, 0.0)
  out_o[...] = _dot(h2, fcW[...]) + fcb[...]


def kernel(x, edge_index, W1_0, W1_1, W1_2, b1_0, b1_1, b1_2,
           W2_0, W2_1, W2_2, b2_0, b2_1, b2_2, fc_W, fc_b):
  n, d_in = x.shape
  e = edge_index.shape[1]
  src = edge_index[0].astype(jnp.int32)
  dst = edge_index[1].astype(jnp.int32)

  ch = None
  for c in range(128, 0, -1):
    if e % (NW * c) == 0:
      ch = c
      break
  if ch is not None:
    pad_rows = 0
    ep = e
  else:
    ch = 64
    ep = -(-e // (NW * ch)) * (NW * ch)
    pad_rows = 16
    fill = jnp.full((ep - e,), n, jnp.int32)
    src = jnp.concatenate([src, fill])
    dst = jnp.concatenate([dst, fill])
  nchunk = ep // (NW * ch)
  src_r = src.reshape(NW, nchunk, ch)
  dst_r = dst.reshape(NW, nchunk, ch)

  def maybe_pad(z):
    if pad_rows:
      return jnp.concatenate(
          [z, jnp.zeros((pad_rows, z.shape[1]), z.dtype)], axis=0)
    return z

  deg_k = _make_sc_scatter(n, pad_rows, 16, nchunk, ch, n_z=0)
  sc2 = _make_sc_scatter(n, pad_rows, 64, nchunk, ch, n_z=2)
  sc1 = _make_sc_scatter(n, pad_rows, 64, nchunk, ch, n_z=1)

  degp = deg_k(src_r, dst_r)

  b10 = b1_0.reshape(1, -1)
  b11 = b1_1.reshape(1, -1)
  b12 = b1_2.reshape(1, -1)
  b20 = b2_0.reshape(1, -1)
  b21 = b2_1.reshape(1, -1)
  b22 = b2_2.reshape(1, -1)
  fcb = fc_b.reshape(1, -1)

  dinv, z1a, z1b = _tc_call(
      _tc1_body, n, (1, 64, 64),
      (degp, x), (_prows(1, 16), _rows(d_in)))

  sp = sc2(maybe_pad(z1a), maybe_pad(z1b), src_r, dst_r)
  u1, z2a, z2b = _tc_call(
      _tc2_body, n, (d_in, 64, 64),
      (sp, z1a, z1b, dinv),
      (_prows(2, 64), _rows(64), _rows(64), _rows(1)))

  sp = sc2(maybe_pad(z2a), maybe_pad(z2b), src_r, dst_r)
  q0, ga, gb = _tc_call(
      _tc3_body, n, (60, 64, 64),
      (sp, z2a, z2b, dinv, x, u1,
       W1_0, b10, W1_1, b11, W1_2, b12, W2_0, b20, W2_1, W2_2),
      (_prows(2, 64), _rows(64), _rows(64), _rows(1), _rows(d_in),
       _rows(d_in),
       _full2(W1_0), _full2(b10), _full2(W1_1), _full2(b11),
       _full2(W1_2), _full2(b12), _full2(W2_0), _full2(b20),
       _full2(W2_1), _full2(W2_2)))

  sp = sc2(maybe_pad(ga), maybe_pad(gb), src_r, dst_r)
  v1, z4 = _tc_call(
      _tc4_body, n, (64, 64),
      (sp, ga, gb, dinv),
      (_prows(2, 64), _rows(64), _rows(64), _rows(1)))

  sp = sc1(maybe_pad(z4), src_r, dst_r)
  (out,) = _tc_call(
      _tc5_body, n, (128,),
      (sp, z4, dinv, q0, v1, b21, b22, fc_W, fcb),
      (_prows(1, 64), _rows(64), _rows(1), _rows(60), _rows(64),
       _full2(b21), _full2(b22), _full2(fc_W), _full2(fcb)))
  return out


# speedup vs baseline: 27.4504x; 1.0049x over previous
import jax
import jax.numpy as jnp
from jax import lax
from jax.experimental import pallas as pl
from jax.experimental.pallas import tpu as pltpu
from jax.experimental.pallas import tpu_sc as plsc

NC = 2
NS = 16
NW = NC * NS
LANES = 16
BR = 1000


def _make_sc_scatter(n_rows, pad_rows, W, nchunk, ch, n_z):
  NZ = 128
  acc_rows = -(-(n_rows + pad_rows) // NZ) * NZ
  nzch = acc_rows // NZ
  wb = next(c for c in range(128, 0, -8) if n_rows % c == 0)
  nwb = n_rows // wb
  npass = max(n_z, 1)

  mesh = plsc.VectorSubcoreMesh(
      core_axis_name="c", subcore_axis_name="s",
      num_cores=NC, num_subcores=NS)

  NB = 2
  if n_z:
    for cand in (4,):
      if nchunk % cand == 0 and nchunk >= 2 * cand:
        NB = cand
        break
  scratch = [
      pltpu.VMEM((nchunk, ch), jnp.int32),
      pltpu.VMEM((nchunk, ch), jnp.int32),
  ] + [pltpu.VMEM((ch, W), jnp.float32) for _ in range(NB)] + [
      pltpu.VMEM((NZ, W), jnp.float32),
      pltpu.VMEM_SHARED((acc_rows, W), jnp.float32),
  ] + [pltpu.SemaphoreType.DMA for _ in range(2 * NB)]

  def body(*refs):
    z_refs = refs[:n_z]
    src_hbm, dst_hbm, out_hbm = refs[n_z:n_z + 3]
    src_v, dst_v = refs[n_z + 3:n_z + 5]
    bufs = refs[n_z + 5:n_z + 5 + NB]
    zbuf = refs[n_z + 5 + NB]
    acc = refs[n_z + 6 + NB]
    gsems = refs[n_z + 7 + NB:n_z + 7 + 2 * NB]
    ssems = refs[n_z + 7 + 2 * NB:]
    bufa, sema = bufs[0], gsems[0]
    c = lax.axis_index("c")
    s = lax.axis_index("s")
    wid = c * NS + s

    @pl.loop(0, NZ)
    def _(i):
      @pl.loop(0, W // LANES)
      def _(k):
        zbuf[i, pl.ds(k * LANES, LANES)] = jnp.zeros((LANES,), jnp.float32)

    pltpu.sync_copy(dst_hbm.at[wid], dst_v)
    if n_z:
      pltpu.sync_copy(src_hbm.at[wid], src_v)
    else:
      @pl.loop(0, ch)
      def _(i):
        @pl.loop(0, W // LANES)
        def _(k):
          bufa[i, pl.ds(k * LANES, LANES)] = jnp.ones((LANES,), jnp.float32)

    for p in range(npass):
      @pl.loop(0, -(-nzch // NS))
      def _(k):
        idx = k * NS + s

        @pl.when(idx < nzch)
        def _():
          pltpu.sync_copy(zbuf, acc.at[pl.ds(idx * NZ, NZ)])

      plsc.subcore_barrier()

      if n_z:
        z_hbm = z_refs[p]
        if NB > 2:
          for k in range(NB):
            pltpu.async_copy(z_hbm.at[src_v.at[k]], bufs[k], gsems[k])

          @pl.loop(0, nchunk // NB)
          def _(j):
            base = NB * j
            for k in range(NB):
              pltpu.make_async_copy(
                  z_hbm.at[src_v.at[base + k]], bufs[k], gsems[k]).wait()
              pltpu.async_copy(
                  bufs[k], acc.at[dst_v.at[base + k]], ssems[k], add=True)
            for k in range(NB):
              pltpu.make_async_copy(
                  bufs[k], acc.at[dst_v.at[base + k]], ssems[k]).wait()

              @pl.when(base + NB + k < nchunk)
              def _(k=k):
                pltpu.async_copy(
                    z_hbm.at[src_v.at[base + NB + k]], bufs[k], gsems[k])
        else:
          bufb, semb = bufs[1], gsems[1]
          pltpu.async_copy(z_hbm.at[src_v.at[0]], bufa, sema)

          @pl.loop(0, nchunk // 2)
          def _(j):
            pltpu.make_async_copy(
                z_hbm.at[src_v.at[2 * j]], bufa, sema).wait()
            pltpu.async_copy(z_hbm.at[src_v.at[2 * j + 1]], bufb, semb)
            pltpu.sync_copy(bufa, acc.at[dst_v.at[2 * j]], add=True)
            pltpu.make_async_copy(
                z_hbm.at[src_v.at[2 * j + 1]], bufb, semb).wait()

            @pl.when(2 * j + 2 < nchunk)
            def _():
              pltpu.async_copy(z_hbm.at[src_v.at[2 * j + 2]], bufa, sema)

            pltpu.sync_copy(bufb, acc.at[dst_v.at[2 * j + 1]], add=True)

          if nchunk % 2 == 1:
            pltpu.make_async_copy(
                z_hbm.at[src_v.at[nchunk - 1]], bufa, sema).wait()
            pltpu.sync_copy(bufa, acc.at[dst_v.at[nchunk - 1]], add=True)
      elif nchunk % 8 == 0:
        @pl.loop(0, nchunk // 8)
        def _(j):
          for k in range(8):
            pltpu.async_copy(bufa, acc.at[dst_v.at[8 * j + k]],
                             ssems[0], add=True)
          for k in range(8):
            pltpu.make_async_copy(bufa, acc.at[dst_v.at[8 * j + k]],
                                  ssems[0]).wait()
      else:
        @pl.loop(0, nchunk)
        def _(j):
          pltpu.sync_copy(bufa, acc.at[dst_v.at[j]], add=True)

      plsc.subcore_barrier()

      @pl.loop(0, -(-nwb // NS))
      def _(k):
        idx = k * NS + s

        @pl.when(idx < nwb)
        def _():
          pltpu.sync_copy(acc.at[pl.ds(idx * wb, wb)],
                          out_hbm.at[c, p, pl.ds(idx * wb, wb)])

      if p + 1 < npass:
        plsc.subcore_barrier()

  return pl.kernel(
      body,
      out_type=jax.ShapeDtypeStruct((NC, npass, n_rows, W), jnp.float32),
      mesh=mesh,
      scratch_types=scratch,
      compiler_params=pltpu.CompilerParams(use_tc_tiling_on_sc=False),
  )


def _rows(w):
  return pl.BlockSpec((BR, w), lambda i: (i, 0))


def _prows(p, w):
  return pl.BlockSpec((NC, p, BR, w), lambda i: (0, 0, i, 0))


def _full2(a):
  return pl.BlockSpec(a.shape, lambda i: (0, 0))


def _tc_call(body, n, out_ws, in_arrays, in_specs):
  return pl.pallas_call(
      body,
      grid=(n // BR,),
      in_specs=in_specs,
      out_specs=tuple(_rows(w) for w in out_ws),
      out_shape=tuple(
          jax.ShapeDtypeStruct((n, w), jnp.float32) for w in out_ws),
  )(*in_arrays)


def _dot(a, b):
  return jnp.dot(a, b, preferred_element_type=jnp.float32)


def _tc1_body(dp, x, dinv_o, z1a_o, z1b_o):
  deg = dp[0, 0, :, :1] + dp[1, 0, :, :1] + 1.0
  dinv = lax.rsqrt(deg)
  dinv_o[...] = dinv
  z1a_o[...] = dinv * x[:, :64]
  z1b_o[...] = dinv * x[:, 64:]


def _tc2_body(sp, z1a, z1b, dinv, u1_o, z2a_o, z2b_o):
  dv = dinv[...]
  ua = dv * (sp[0, 0] + sp[1, 0] + z1a[...])
  ub = dv * (sp[0, 1] + sp[1, 1] + z1b[...])
  u1_o[...] = jnp.concatenate([ua, ub], axis=1)
  z2a_o[...] = dv * ua
  z2b_o[...] = dv * ub


def _tc3_body(sp, z2a, z2b, dinv, x, u1, W10, b10, W11, b11, W12, b12,
              W20, b20, W21, W22, q0_o, ga_o, gb_o):
  dv = dinv[...]
  u2 = dv * jnp.concatenate(
      [sp[0, 0] + sp[1, 0] + z2a[...], sp[0, 1] + sp[1, 1] + z2b[...]],
      axis=1)
  h = jnp.concatenate([
      _dot(x[...], W10[...]) + b10[...],
      _dot(u1[...], W11[...]) + b11[...],
      _dot(u2, W12[...]) + b12[...],
  ], axis=1)
  h = jnp.maximum(h, 0.0)
  q0_o[...] = _dot(h, W20[...]) + b20[...]
  zpad = jnp.zeros((h.shape[0], 4), jnp.float32)
  ga_o[...] = dv * jnp.concatenate([_dot(h, W21[...]), zpad], axis=1)
  gb_o[...] = dv * jnp.concatenate([_dot(h, W22[...]), zpad], axis=1)


def _tc4_body(sp, ga, gb, dinv, v1_o, z4_o):
  dv = dinv[...]
  v1_o[...] = dv * (sp[0, 0] + sp[1, 0] + ga[...])
  z4_o[...] = dv * dv * (sp[0, 1] + sp[1, 1] + gb[...])


def _tc5_body(sp, z4, dinv, q0, v1, b21, b22, fcW, fcb, out_o):
  dv = dinv[...]
  w = dv * (sp[0, 0] + sp[1, 0] + z4[...])
  h2 = jnp.concatenate([
      q0[...],
      v1[:, :60] + b21[...],
      w[:, :60] + b22[...],
  ], axis=1)
  h2 = jnp.maximum(h2, 0.0)
  out_o[...] = _dot(h2, fcW[...]) + fcb[...]


def kernel(x, edge_index, W1_0, W1_1, W1_2, b1_0, b1_1, b1_2,
           W2_0, W2_1, W2_2, b2_0, b2_1, b2_2, fc_W, fc_b):
  n, d_in = x.shape
  e = edge_index.shape[1]
  src = edge_index[0].astype(jnp.int32)
  dst = edge_index[1].astype(jnp.int32)

  ch = None
  for c in range(128, 0, -1):
    if e % (NW * c) == 0:
      ch = c
      break
  if ch is not None:
    pad_rows = 0
    ep = e
  else:
    ch = 64
    ep = -(-e // (NW * ch)) * (NW * ch)
    pad_rows = 16
    fill = jnp.full((ep - e,), n, jnp.int32)
    src = jnp.concatenate([src, fill])
    dst = jnp.concatenate([dst, fill])
  nchunk = ep // (NW * ch)
  src_r = src.reshape(NW, nchunk, ch)
  dst_r = dst.reshape(NW, nchunk, ch)

  def maybe_pad(z):
    if pad_rows:
      return jnp.concatenate(
          [z, jnp.zeros((pad_rows, z.shape[1]), z.dtype)], axis=0)
    return z

  deg_k = _make_sc_scatter(n, pad_rows, 16, nchunk, ch, n_z=0)
  sc2 = _make_sc_scatter(n, pad_rows, 64, nchunk, ch, n_z=2)
  sc1 = _make_sc_scatter(n, pad_rows, 64, nchunk, ch, n_z=1)

  degp = deg_k(src_r, dst_r)

  b10 = b1_0.reshape(1, -1)
  b11 = b1_1.reshape(1, -1)
  b12 = b1_2.reshape(1, -1)
  b20 = b2_0.reshape(1, -1)
  b21 = b2_1.reshape(1, -1)
  b22 = b2_2.reshape(1, -1)
  fcb = fc_b.reshape(1, -1)

  dinv, z1a, z1b = _tc_call(
      _tc1_body, n, (1, 64, 64),
      (degp, x), (_prows(1, 16), _rows(d_in)))

  sp = sc2(maybe_pad(z1a), maybe_pad(z1b), src_r, dst_r)
  u1, z2a, z2b = _tc_call(
      _tc2_body, n, (d_in, 64, 64),
      (sp, z1a, z1b, dinv),
      (_prows(2, 64), _rows(64), _rows(64), _rows(1)))

  sp = sc2(maybe_pad(z2a), maybe_pad(z2b), src_r, dst_r)
  q0, ga, gb = _tc_call(
      _tc3_body, n, (60, 64, 64),
      (sp, z2a, z2b, dinv, x, u1,
       W1_0, b10, W1_1, b11, W1_2, b12, W2_0, b20, W2_1, W2_2),
      (_prows(2, 64), _rows(64), _rows(64), _rows(1), _rows(d_in),
       _rows(d_in),
       _full2(W1_0), _full2(b10), _full2(W1_1), _full2(b11),
       _full2(W1_2), _full2(b12), _full2(W2_0), _full2(b20),
       _full2(W2_1), _full2(W2_2)))

  sp = sc2(maybe_pad(ga), maybe_pad(gb), src_r, dst_r)
  v1, z4 = _tc_call(
      _tc4_body, n, (64, 64),
      (sp, ga, gb, dinv),
      (_prows(2, 64), _rows(64), _rows(64), _rows(1)))

  sp = sc1(maybe_pad(z4), src_r, dst_r)
  (out,) = _tc_call(
      _tc5_body, n, (128,),
      (sp, z4, dinv, q0, v1, b21, b22, fc_W, fcb),
      (_prows(1, 64), _rows(64), _rows(1), _rows(60), _rows(64),
       _full2(b21), _full2(b22), _full2(fc_W), _full2(fcb)))
  return out


# speedup vs baseline: 32.4590x; 1.1825x over previous
import jax
import jax.numpy as jnp
from jax import lax
from jax.experimental import pallas as pl
from jax.experimental.pallas import tpu as pltpu
from jax.experimental.pallas import tpu_sc as plsc

NC = 2
NS = 16
NW = NC * NS
LANES = 16
BR = 1000


def _make_sc_scatter(n_rows, pad_rows, W, nchunk, ch, npass, deg_mode=False):
  NZ = 128
  acc_rows = -(-(n_rows + pad_rows) // NZ) * NZ
  nzch = acc_rows // NZ
  wb = next(c for c in range(128, 0, -8) if n_rows % c == 0)
  nwb = n_rows // wb
  out_w = W if deg_mode else 128

  mesh = plsc.VectorSubcoreMesh(
      core_axis_name="c", subcore_axis_name="s",
      num_cores=NC, num_subcores=NS)

  NB = 2
  if not deg_mode:
    for cand in (4,):
      if nchunk % cand == 0 and nchunk >= 2 * cand:
        NB = cand
        break

  scratch = [
      pltpu.VMEM((nchunk, ch), jnp.int32),
      pltpu.VMEM((nchunk, ch), jnp.int32),
  ] + [pltpu.VMEM((ch, W), jnp.float32) for _ in range(NB)] + [
      pltpu.VMEM((NZ, W), jnp.float32),
      pltpu.VMEM_SHARED((acc_rows, W), jnp.float32),
  ] + [pltpu.SemaphoreType.DMA for _ in range(2 * NB)]

  n_src = 0 if deg_mode else npass

  def body(*refs):
    z_hbm = None if deg_mode else refs[0]
    src_refs = refs[1:1 + n_src] if not deg_mode else ()
    k0 = (1 + n_src) if not deg_mode else 0
    dst_hbm, out_hbm = refs[k0:k0 + 2]
    src_v, dst_v = refs[k0 + 2:k0 + 4]
    bufs = refs[k0 + 4:k0 + 4 + NB]
    zbuf = refs[k0 + 4 + NB]
    acc = refs[k0 + 5 + NB]
    gsems = refs[k0 + 6 + NB:k0 + 6 + 2 * NB]
    ssems = refs[k0 + 6 + 2 * NB:]
    bufa, sema = bufs[0], gsems[0]
    c = lax.axis_index("c")
    s = lax.axis_index("s")
    wid = c * NS + s

    @pl.loop(0, NZ)
    def _(i):
      @pl.loop(0, W // LANES)
      def _(k):
        zbuf[i, pl.ds(k * LANES, LANES)] = jnp.zeros((LANES,), jnp.float32)

    pltpu.sync_copy(dst_hbm.at[wid], dst_v)
    if deg_mode:
      @pl.loop(0, ch)
      def _(i):
        @pl.loop(0, W // LANES)
        def _(k):
          bufa[i, pl.ds(k * LANES, LANES)] = jnp.ones((LANES,), jnp.float32)

    for p in range(npass):
      @pl.loop(0, -(-nzch // NS))
      def _(k):
        idx = k * NS + s

        @pl.when(idx < nzch)
        def _():
          pltpu.sync_copy(zbuf, acc.at[pl.ds(idx * NZ, NZ)])

      if not deg_mode:
        pltpu.sync_copy(src_refs[p].at[wid], src_v)

      plsc.subcore_barrier()

      if not deg_mode:
        if NB > 2:
          for k in range(NB):
            pltpu.async_copy(z_hbm.at[src_v.at[k]], bufs[k], gsems[k])

          @pl.loop(0, nchunk // NB)
          def _(j):
            base = NB * j
            for k in range(NB):
              pltpu.make_async_copy(
                  z_hbm.at[src_v.at[base + k]], bufs[k], gsems[k]).wait()
              pltpu.async_copy(
                  bufs[k], acc.at[dst_v.at[base + k]], ssems[k], add=True)
            for k in range(NB):
              pltpu.make_async_copy(
                  bufs[k], acc.at[dst_v.at[base + k]], ssems[k]).wait()

              @pl.when(base + NB + k < nchunk)
              def _(k=k):
                pltpu.async_copy(
                    z_hbm.at[src_v.at[base + NB + k]], bufs[k], gsems[k])
        else:
          bufb, semb = bufs[1], gsems[1]
          pltpu.async_copy(z_hbm.at[src_v.at[0]], bufa, sema)

          @pl.loop(0, nchunk // 2)
          def _(j):
            pltpu.make_async_copy(
                z_hbm.at[src_v.at[2 * j]], bufa, sema).wait()
            pltpu.async_copy(z_hbm.at[src_v.at[2 * j + 1]], bufb, semb)
            pltpu.sync_copy(bufa, acc.at[dst_v.at[2 * j]], add=True)
            pltpu.make_async_copy(
                z_hbm.at[src_v.at[2 * j + 1]], bufb, semb).wait()

            @pl.when(2 * j + 2 < nchunk)
            def _():
              pltpu.async_copy(z_hbm.at[src_v.at[2 * j + 2]], bufa, sema)

            pltpu.sync_copy(bufb, acc.at[dst_v.at[2 * j + 1]], add=True)

          if nchunk % 2 == 1:
            pltpu.make_async_copy(
                z_hbm.at[src_v.at[nchunk - 1]], bufa, sema).wait()
            pltpu.sync_copy(bufa, acc.at[dst_v.at[nchunk - 1]], add=True)
      elif nchunk % 8 == 0:
        @pl.loop(0, nchunk // 8)
        def _(j):
          for k in range(8):
            pltpu.async_copy(bufa, acc.at[dst_v.at[8 * j + k]],
                             ssems[0], add=True)
          for k in range(8):
            pltpu.make_async_copy(bufa, acc.at[dst_v.at[8 * j + k]],
                                  ssems[0]).wait()
      else:
        @pl.loop(0, nchunk)
        def _(j):
          pltpu.sync_copy(bufa, acc.at[dst_v.at[j]], add=True)

      plsc.subcore_barrier()

      @pl.loop(0, -(-nwb // NS))
      def _(k):
        idx = k * NS + s

        @pl.when(idx < nwb)
        def _():
          if deg_mode:
            pltpu.sync_copy(acc.at[pl.ds(idx * wb, wb)],
                            out_hbm.at[c, pl.ds(idx * wb, wb)])
          else:
            pltpu.sync_copy(acc.at[pl.ds(idx * wb, wb)],
                            out_hbm.at[c, pl.ds(idx * wb, wb),
                                       pl.ds(p * W, W)])

      if p + 1 < npass:
        plsc.subcore_barrier()

  return pl.kernel(
      body,
      out_type=jax.ShapeDtypeStruct((NC, n_rows, out_w), jnp.float32),
      mesh=mesh,
      scratch_types=scratch,
      compiler_params=pltpu.CompilerParams(use_tc_tiling_on_sc=False),
  )


def _rows(w):
  return pl.BlockSpec((BR, w), lambda i: (i, 0))


def _prows(w):
  return pl.BlockSpec((NC, BR, w), lambda i: (0, i, 0))


def _full2(a):
  return pl.BlockSpec(a.shape, lambda i: (0, 0))


def _tc_call(body, n, out_ws, in_arrays, in_specs):
  return pl.pallas_call(
      body,
      grid=(n // BR,),
      in_specs=in_specs,
      out_specs=tuple(_rows(w) for w in out_ws),
      out_shape=tuple(
          jax.ShapeDtypeStruct((n, w), jnp.float32) for w in out_ws),
  )(*in_arrays)


def _dot(a, b):
  return jnp.dot(a, b, preferred_element_type=jnp.float32)


def _tc1_body(dp, x, dinv_o, z1_o):
  deg = dp[0, :, :1] + dp[1, :, :1] + 1.0
  dinv = lax.rsqrt(deg)
  dinv_o[...] = dinv
  z1_o[...] = dinv * x[...]


def _tc2_body(sp, z1, dinv, u1_o, z2_o):
  dv = dinv[...]
  u1 = dv * (sp[0] + sp[1] + z1[...])
  u1_o[...] = u1
  z2_o[...] = dv * u1


def _tc3_body(sp, z2, dinv, x, u1, W10, b10, W11, b11, W12, b12,
              W20, b20, W21, W22, q0_o, g_o):
  dv = dinv[...]
  u2 = dv * (sp[0] + sp[1] + z2[...])
  h = jnp.concatenate([
      _dot(x[...], W10[...]) + b10[...],
      _dot(u1[...], W11[...]) + b11[...],
      _dot(u2, W12[...]) + b12[...],
  ], axis=1)
  h = jnp.maximum(h, 0.0)
  q0_o[...] = _dot(h, W20[...]) + b20[...]
  zpad = jnp.zeros((h.shape[0], 4), jnp.float32)
  g_o[...] = dv * jnp.concatenate(
      [_dot(h, W21[...]), zpad, _dot(h, W22[...]), zpad], axis=1)


def _tc4_body(sp, g, dinv, vz_o):
  dv = dinv[...]
  v = dv * (sp[0] + sp[1] + g[...])
  vz_o[...] = jnp.concatenate([dv * v[:, 64:], v[:, :64]], axis=1)


def _tc5_body(sp, vz, dinv, q0, b21, b22, fcW, fcb, out_o):
  dv = dinv[...]
  z4 = vz[:, :64]
  w = dv * (sp[0, :, :64] + sp[1, :, :64] + z4)
  h2 = jnp.concatenate([
      q0[...],
      vz[:, 64:124] + b21[...],
      w[:, :60] + b22[...],
  ], axis=1)
  h2 = jnp.maximum(h2, 0.0)
  out_o[...] = _dot(h2, fcW[...]) + fcb[...]


def kernel(x, edge_index, W1_0, W1_1, W1_2, b1_0, b1_1, b1_2,
           W2_0, W2_1, W2_2, b2_0, b2_1, b2_2, fc_W, fc_b):
  n, d_in = x.shape
  e = edge_index.shape[1]
  src = edge_index[0].astype(jnp.int32)
  dst = edge_index[1].astype(jnp.int32)

  ch = None
  for c in range(128, 0, -1):
    if e % (NW * c) == 0:
      ch = c
      break
  if ch is not None:
    pad_rows = 0
    ep = e
  else:
    ch = 64
    ep = -(-e // (NW * ch)) * (NW * ch)
    pad_rows = 16
    fill = jnp.full((ep - e,), n, jnp.int32)
    src = jnp.concatenate([src, fill])
    dst = jnp.concatenate([dst, fill])
  nchunk = ep // (NW * ch)
  src_a = (2 * src).reshape(NW, nchunk, ch)
  src_b = (2 * src + 1).reshape(NW, nchunk, ch)
  dst_r = dst.reshape(NW, nchunk, ch)

  def flat(z):
    if pad_rows:
      z = jnp.concatenate(
          [z, jnp.zeros((pad_rows, z.shape[1]), z.dtype)], axis=0)
    return z.reshape(2 * z.shape[0], 64)

  deg_k = _make_sc_scatter(n, pad_rows, 16, nchunk, ch, 1, deg_mode=True)
  sc2 = _make_sc_scatter(n, pad_rows, 64, nchunk, ch, 2)
  sc1 = _make_sc_scatter(n, pad_rows, 64, nchunk, ch, 1)

  degp = deg_k(dst_r)

  b10 = b1_0.reshape(1, -1)
  b11 = b1_1.reshape(1, -1)
  b12 = b1_2.reshape(1, -1)
  b20 = b2_0.reshape(1, -1)
  b21 = b2_1.reshape(1, -1)
  b22 = b2_2.reshape(1, -1)
  fcb = fc_b.reshape(1, -1)

  dinv, z1 = _tc_call(
      _tc1_body, n, (1, d_in),
      (degp, x), (_prows(16), _rows(d_in)))

  sp = sc2(flat(z1), src_a, src_b, dst_r)
  u1, z2 = _tc_call(
      _tc2_body, n, (d_in, d_in),
      (sp, z1, dinv), (_prows(128), _rows(128), _rows(1)))

  sp = sc2(flat(z2), src_a, src_b, dst_r)
  q0, g = _tc_call(
      _tc3_body, n, (60, 128),
      (sp, z2, dinv, x, u1,
       W1_0, b10, W1_1, b11, W1_2, b12, W2_0, b20, W2_1, W2_2),
      (_prows(128), _rows(128), _rows(1), _rows(d_in), _rows(d_in),
       _full2(W1_0), _full2(b10), _full2(W1_1), _full2(b11),
       _full2(W1_2), _full2(b12), _full2(W2_0), _full2(b20),
       _full2(W2_1), _full2(W2_2)))

  sp = sc2(flat(g), src_a, src_b, dst_r)
  (vz,) = _tc_call(
      _tc4_body, n, (128,),
      (sp, g, dinv), (_prows(128), _rows(128), _rows(1)))

  sp = sc1(flat(vz), src_a, dst_r)
  (out,) = _tc_call(
      _tc5_body, n, (128,),
      (sp, vz, dinv, q0, b21, b22, fc_W, fcb),
      (_prows(128), _rows(128), _rows(1), _rows(60),
       _full2(b21), _full2(b22), _full2(fc_W), _full2(fcb)))
  return out


# speedup vs baseline: 33.4090x; 1.0293x over previous
import jax
import jax.numpy as jnp
from jax import lax
from jax.experimental import pallas as pl
from jax.experimental.pallas import tpu as pltpu
from jax.experimental.pallas import tpu_sc as plsc

NC = 2
NS = 16
NW = NC * NS
LANES = 16
BR = 1000


def _make_sc_scatter(n_rows, pad_rows, W, nchunk, ch, npass, deg_mode=False):
  NZ = 128
  acc_rows = -(-(n_rows + pad_rows) // NZ) * NZ
  nzch = acc_rows // NZ
  wb = next(c for c in range(128, 0, -8) if n_rows % c == 0)
  nwb = n_rows // wb
  out_w = 128

  mesh = plsc.VectorSubcoreMesh(
      core_axis_name="c", subcore_axis_name="s",
      num_cores=NC, num_subcores=NS)

  NB = 2
  if not deg_mode:
    for cand in (5, 4):
      if nchunk % cand == 0 and nchunk >= 2 * cand:
        NB = cand
        break

  scratch = [
      pltpu.VMEM((nchunk, ch), jnp.int32),
      pltpu.VMEM((nchunk, ch), jnp.int32),
  ] + [pltpu.VMEM((ch, W), jnp.float32) for _ in range(NB)] + [
      pltpu.VMEM((NZ, W), jnp.float32),
      pltpu.VMEM_SHARED((acc_rows, W), jnp.float32),
  ] + [pltpu.SemaphoreType.DMA for _ in range(2 * NB)]

  n_src = 0 if deg_mode else npass

  def body(*refs):
    z_hbm = None if deg_mode else refs[0]
    src_refs = refs[1:1 + n_src] if not deg_mode else ()
    k0 = (1 + n_src) if not deg_mode else 0
    dst_hbm, out_hbm = refs[k0:k0 + 2]
    src_v, dst_v = refs[k0 + 2:k0 + 4]
    bufs = refs[k0 + 4:k0 + 4 + NB]
    zbuf = refs[k0 + 4 + NB]
    acc = refs[k0 + 5 + NB]
    gsems = refs[k0 + 6 + NB:k0 + 6 + 2 * NB]
    ssems = refs[k0 + 6 + 2 * NB:]
    bufa, sema = bufs[0], gsems[0]
    c = lax.axis_index("c")
    s = lax.axis_index("s")
    wid = c * NS + s

    @pl.loop(0, NZ)
    def _(i):
      @pl.loop(0, W // LANES)
      def _(k):
        zbuf[i, pl.ds(k * LANES, LANES)] = jnp.zeros((LANES,), jnp.float32)

    pltpu.sync_copy(dst_hbm.at[wid], dst_v)
    if deg_mode:
      @pl.loop(0, ch)
      def _(i):
        @pl.loop(0, W // LANES)
        def _(k):
          bufa[i, pl.ds(k * LANES, LANES)] = jnp.ones((LANES,), jnp.float32)

    for p in range(npass):
      @pl.loop(0, -(-nzch // NS))
      def _(k):
        idx = k * NS + s

        @pl.when(idx < nzch)
        def _():
          pltpu.sync_copy(zbuf, acc.at[pl.ds(idx * NZ, NZ)])

      if not deg_mode:
        pltpu.sync_copy(src_refs[p].at[wid], src_v)

      plsc.subcore_barrier()

      if not deg_mode:
        if NB > 2:
          for k in range(NB):
            pltpu.async_copy(z_hbm.at[src_v.at[k]], bufs[k], gsems[k])

          @pl.loop(0, nchunk // NB)
          def _(j):
            base = NB * j
            for k in range(NB):
              pltpu.make_async_copy(
                  z_hbm.at[src_v.at[base + k]], bufs[k], gsems[k]).wait()
              pltpu.async_copy(
                  bufs[k], acc.at[dst_v.at[base + k]], ssems[k], add=True)
            for k in range(NB):
              pltpu.make_async_copy(
                  bufs[k], acc.at[dst_v.at[base + k]], ssems[k]).wait()

              @pl.when(base + NB + k < nchunk)
              def _(k=k):
                pltpu.async_copy(
                    z_hbm.at[src_v.at[base + NB + k]], bufs[k], gsems[k])
        else:
          bufb, semb = bufs[1], gsems[1]
          pltpu.async_copy(z_hbm.at[src_v.at[0]], bufa, sema)

          @pl.loop(0, nchunk // 2)
          def _(j):
            pltpu.make_async_copy(
                z_hbm.at[src_v.at[2 * j]], bufa, sema).wait()
            pltpu.async_copy(z_hbm.at[src_v.at[2 * j + 1]], bufb, semb)
            pltpu.sync_copy(bufa, acc.at[dst_v.at[2 * j]], add=True)
            pltpu.make_async_copy(
                z_hbm.at[src_v.at[2 * j + 1]], bufb, semb).wait()

            @pl.when(2 * j + 2 < nchunk)
            def _():
              pltpu.async_copy(z_hbm.at[src_v.at[2 * j + 2]], bufa, sema)

            pltpu.sync_copy(bufb, acc.at[dst_v.at[2 * j + 1]], add=True)

          if nchunk % 2 == 1:
            pltpu.make_async_copy(
                z_hbm.at[src_v.at[nchunk - 1]], bufa, sema).wait()
            pltpu.sync_copy(bufa, acc.at[dst_v.at[nchunk - 1]], add=True)
      elif nchunk % 8 == 0:
        @pl.loop(0, nchunk // 8)
        def _(j):
          for k in range(8):
            pltpu.async_copy(bufa, acc.at[dst_v.at[8 * j + k]],
                             ssems[0], add=True)
          for k in range(8):
            pltpu.make_async_copy(bufa, acc.at[dst_v.at[8 * j + k]],
                                  ssems[0]).wait()
      else:
        @pl.loop(0, nchunk)
        def _(j):
          pltpu.sync_copy(bufa, acc.at[dst_v.at[j]], add=True)

      plsc.subcore_barrier()

      @pl.loop(0, -(-nwb // NS))
      def _(k):
        idx = k * NS + s

        @pl.when(idx < nwb)
        def _():
          pltpu.sync_copy(acc.at[pl.ds(idx * wb, wb)],
                          out_hbm.at[c, pl.ds(idx * wb, wb),
                                     pl.ds(p * W, W)])

      if p + 1 < npass:
        plsc.subcore_barrier()

  return pl.kernel(
      body,
      out_type=jax.ShapeDtypeStruct((NC, n_rows, out_w), jnp.float32),
      mesh=mesh,
      scratch_types=scratch,
      compiler_params=pltpu.CompilerParams(use_tc_tiling_on_sc=False),
  )


def _rows(w):
  return pl.BlockSpec((BR, w), lambda i: (i, 0))


def _prows(w):
  return pl.BlockSpec((NC, BR, w), lambda i: (0, i, 0))


def _full2(a):
  return pl.BlockSpec(a.shape, lambda i: (0, 0))


def _tc_call(body, n, out_ws, in_arrays, in_specs):
  return pl.pallas_call(
      body,
      grid=(n // BR,),
      in_specs=in_specs,
      out_specs=tuple(_rows(w) for w in out_ws),
      out_shape=tuple(
          jax.ShapeDtypeStruct((n, w), jnp.float32) for w in out_ws),
  )(*in_arrays)


def _dot(a, b):
  return jnp.dot(a, b, preferred_element_type=jnp.float32)


def _tc1_body(dp, x, dinv_o, z1_o):
  deg = dp[0, :, :1] + dp[1, :, :1] + 1.0
  dinv = lax.rsqrt(deg)
  dinv_o[...] = dinv
  z1_o[...] = dinv * x[...]


def _tc2_body(sp, z1, dinv, u1_o, z2_o):
  dv = dinv[...]
  u1 = dv * (sp[0] + sp[1] + z1[...])
  u1_o[...] = u1
  z2_o[...] = dv * u1


def _tc3_body(sp, z2, dinv, x, u1, W10, b10, W11, b11, W12, b12,
              W20, b20, W21, W22, q0_o, g_o):
  dv = dinv[...]
  u2 = dv * (sp[0] + sp[1] + z2[...])
  h = jnp.concatenate([
      _dot(x[...], W10[...]) + b10[...],
      _dot(u1[...], W11[...]) + b11[...],
      _dot(u2, W12[...]) + b12[...],
  ], axis=1)
  h = jnp.maximum(h, 0.0)
  q0_o[...] = _dot(h, W20[...]) + b20[...]
  zpad = jnp.zeros((h.shape[0], 4), jnp.float32)
  g_o[...] = dv * jnp.concatenate(
      [_dot(h, W21[...]), zpad, _dot(h, W22[...]), zpad], axis=1)


def _tc4_body(sp, g, dinv, vz_o):
  dv = dinv[...]
  v = dv * (sp[0] + sp[1] + g[...])
  vz_o[...] = jnp.concatenate([dv * v[:, 64:], v[:, :64]], axis=1)


def _tc5_body(sp, vz, dinv, q0, b21, b22, fcW, fcb, out_o):
  dv = dinv[...]
  z4 = vz[:, :64]
  w = dv * (sp[0, :, :64] + sp[1, :, :64] + z4)
  h2 = jnp.concatenate([
      q0[...],
      vz[:, 64:124] + b21[...],
      w[:, :60] + b22[...],
  ], axis=1)
  h2 = jnp.maximum(h2, 0.0)
  out_o[...] = _dot(h2, fcW[...]) + fcb[...]


def kernel(x, edge_index, W1_0, W1_1, W1_2, b1_0, b1_1, b1_2,
           W2_0, W2_1, W2_2, b2_0, b2_1, b2_2, fc_W, fc_b):
  n, d_in = x.shape
  e = edge_index.shape[1]
  src = edge_index[0].astype(jnp.int32)
  dst = edge_index[1].astype(jnp.int32)

  ch = None
  for c in range(128, 0, -1):
    if e % (NW * c) == 0:
      ch = c
      break
  if ch is not None:
    pad_rows = 0
    ep = e
  else:
    ch = 64
    ep = -(-e // (NW * ch)) * (NW * ch)
    pad_rows = 16
    fill = jnp.full((ep - e,), n, jnp.int32)
    src = jnp.concatenate([src, fill])
    dst = jnp.concatenate([dst, fill])
  nchunk = ep // (NW * ch)
  src_a = (2 * src).reshape(NW, nchunk, ch)
  src_b = (2 * src + 1).reshape(NW, nchunk, ch)
  dst_r = dst.reshape(NW, nchunk, ch)

  def flat(z):
    if pad_rows:
      z = jnp.concatenate(
          [z, jnp.zeros((pad_rows, z.shape[1]), z.dtype)], axis=0)
    return z.reshape(2 * z.shape[0], 64)

  deg_k = _make_sc_scatter(n, pad_rows, 16, nchunk, ch, 1, deg_mode=True)
  sc2 = _make_sc_scatter(n, pad_rows, 64, nchunk, ch, 2)
  sc1 = _make_sc_scatter(n, pad_rows, 64, nchunk, ch, 1)

  degp = deg_k(dst_r)

  b10 = b1_0.reshape(1, -1)
  b11 = b1_1.reshape(1, -1)
  b12 = b1_2.reshape(1, -1)
  b20 = b2_0.reshape(1, -1)
  b21 = b2_1.reshape(1, -1)
  b22 = b2_2.reshape(1, -1)
  fcb = fc_b.reshape(1, -1)

  dinv, z1 = _tc_call(
      _tc1_body, n, (1, d_in),
      (degp, x), (_prows(128), _rows(d_in)))

  sp = sc2(flat(z1), src_a, src_b, dst_r)
  u1, z2 = _tc_call(
      _tc2_body, n, (d_in, d_in),
      (sp, z1, dinv), (_prows(128), _rows(128), _rows(1)))

  sp = sc2(flat(z2), src_a, src_b, dst_r)
  q0, g = _tc_call(
      _tc3_body, n, (60, 128),
      (sp, z2, dinv, x, u1,
       W1_0, b10, W1_1, b11, W1_2, b12, W2_0, b20, W2_1, W2_2),
      (_prows(128), _rows(128), _rows(1), _rows(d_in), _rows(d_in),
       _full2(W1_0), _full2(b10), _full2(W1_1), _full2(b11),
       _full2(W1_2), _full2(b12), _full2(W2_0), _full2(b20),
       _full2(W2_1), _full2(W2_2)))

  sp = sc2(flat(g), src_a, src_b, dst_r)
  (vz,) = _tc_call(
      _tc4_body, n, (128,),
      (sp, g, dinv), (_prows(128), _rows(128), _rows(1)))

  sp = sc1(flat(vz), src_a, dst_r)
  (out,) = _tc_call(
      _tc5_body, n, (128,),
      (sp, vz, dinv, q0, b21, b22, fc_W, fcb),
      (_prows(128), _rows(128), _rows(1), _rows(60),
       _full2(b21), _full2(b22), _full2(fc_W), _full2(fcb)))
  return out


# speedup vs baseline: 33.5662x; 1.0047x over previous
import jax
import jax.numpy as jnp
from jax import lax
from jax.experimental import pallas as pl
from jax.experimental.pallas import tpu as pltpu
from jax.experimental.pallas import tpu_sc as plsc

NC = 2
NS = 16
NW = NC * NS
LANES = 16
BR = 1000


def _make_sc_scatter(n_rows, pad_rows, W, nchunk, ch, npass, deg_mode=False):
  NZ = 128
  acc_rows = -(-(n_rows + pad_rows) // NZ) * NZ
  nzch = acc_rows // NZ
  wb = next(c for c in range(128, 0, -8) if n_rows % c == 0)
  nwb = n_rows // wb
  out_w = 128

  mesh = plsc.VectorSubcoreMesh(
      core_axis_name="c", subcore_axis_name="s",
      num_cores=NC, num_subcores=NS)

  NB = 2
  if not deg_mode:
    for cand in (5, 4):
      if nchunk % cand == 0 and nchunk >= 2 * cand:
        NB = cand
        break

  scratch = [
      pltpu.VMEM((nchunk, ch), jnp.int32),
      pltpu.VMEM((nchunk, ch), jnp.int32),
  ] + [pltpu.VMEM((ch, W), jnp.float32) for _ in range(NB)] + [
      pltpu.VMEM((NZ, W), jnp.float32),
      pltpu.VMEM_SHARED((acc_rows, W), jnp.float32),
  ] + [pltpu.SemaphoreType.DMA for _ in range(2 * NB)]

  n_src = 0 if deg_mode else npass

  def body(*refs):
    z_hbm = None if deg_mode else refs[0]
    src_refs = refs[1:1 + n_src] if not deg_mode else ()
    k0 = (1 + n_src) if not deg_mode else 0
    dst_hbm, out_hbm = refs[k0:k0 + 2]
    src_v, dst_v = refs[k0 + 2:k0 + 4]
    bufs = refs[k0 + 4:k0 + 4 + NB]
    zbuf = refs[k0 + 4 + NB]
    acc = refs[k0 + 5 + NB]
    gsems = refs[k0 + 6 + NB:k0 + 6 + 2 * NB]
    ssems = refs[k0 + 6 + 2 * NB:]
    bufa, sema = bufs[0], gsems[0]
    c = lax.axis_index("c")
    s = lax.axis_index("s")
    wid = c * NS + s

    @pl.loop(0, NZ)
    def _(i):
      @pl.loop(0, W // LANES)
      def _(k):
        zbuf[i, pl.ds(k * LANES, LANES)] = jnp.zeros((LANES,), jnp.float32)

    pltpu.sync_copy(dst_hbm.at[wid], dst_v)
    if deg_mode:
      @pl.loop(0, ch)
      def _(i):
        @pl.loop(0, W // LANES)
        def _(k):
          bufa[i, pl.ds(k * LANES, LANES)] = jnp.ones((LANES,), jnp.float32)

    for p in range(npass):
      @pl.loop(0, -(-nzch // NS))
      def _(k):
        idx = k * NS + s

        @pl.when(idx < nzch)
        def _():
          pltpu.sync_copy(zbuf, acc.at[pl.ds(idx * NZ, NZ)])

      if not deg_mode:
        pltpu.sync_copy(src_refs[p].at[wid], src_v)

      plsc.subcore_barrier()

      if not deg_mode:
        if NB > 2:
          for k in range(NB):
            pltpu.async_copy(z_hbm.at[src_v.at[k]], bufs[k], gsems[k])

          @pl.loop(0, nchunk // NB)
          def _(j):
            base = NB * j
            for k in range(NB):
              pltpu.make_async_copy(
                  z_hbm.at[src_v.at[base + k]], bufs[k], gsems[k]).wait()
              pltpu.async_copy(
                  bufs[k], acc.at[dst_v.at[base + k]], ssems[k], add=True)
            for k in range(NB):
              pltpu.make_async_copy(
                  bufs[k], acc.at[dst_v.at[base + k]], ssems[k]).wait()

              @pl.when(base + NB + k < nchunk)
              def _(k=k):
                pltpu.async_copy(
                    z_hbm.at[src_v.at[base + NB + k]], bufs[k], gsems[k])
        else:
          bufb, semb = bufs[1], gsems[1]
          pltpu.async_copy(z_hbm.at[src_v.at[0]], bufa, sema)

          @pl.loop(0, nchunk // 2)
          def _(j):
            pltpu.make_async_copy(
                z_hbm.at[src_v.at[2 * j]], bufa, sema).wait()
            pltpu.async_copy(z_hbm.at[src_v.at[2 * j + 1]], bufb, semb)
            pltpu.sync_copy(bufa, acc.at[dst_v.at[2 * j]], add=True)
            pltpu.make_async_copy(
                z_hbm.at[src_v.at[2 * j + 1]], bufb, semb).wait()

            @pl.when(2 * j + 2 < nchunk)
            def _():
              pltpu.async_copy(z_hbm.at[src_v.at[2 * j + 2]], bufa, sema)

            pltpu.sync_copy(bufb, acc.at[dst_v.at[2 * j + 1]], add=True)

          if nchunk % 2 == 1:
            pltpu.make_async_copy(
                z_hbm.at[src_v.at[nchunk - 1]], bufa, sema).wait()
            pltpu.sync_copy(bufa, acc.at[dst_v.at[nchunk - 1]], add=True)
      elif nchunk % 8 == 0:
        @pl.loop(0, nchunk // 8)
        def _(j):
          for k in range(8):
            pltpu.async_copy(bufa, acc.at[dst_v.at[8 * j + k]],
                             ssems[0], add=True)
          for k in range(8):
            pltpu.make_async_copy(bufa, acc.at[dst_v.at[8 * j + k]],
                                  ssems[0]).wait()
      else:
        @pl.loop(0, nchunk)
        def _(j):
          pltpu.sync_copy(bufa, acc.at[dst_v.at[j]], add=True)

      plsc.subcore_barrier()

      @pl.loop(0, -(-nwb // NS))
      def _(k):
        idx = k * NS + s

        @pl.when(idx < nwb)
        def _():
          pltpu.sync_copy(acc.at[pl.ds(idx * wb, wb)],
                          out_hbm.at[c, pl.ds(idx * wb, wb),
                                     pl.ds(p * W, W)])

      if p + 1 < npass:
        plsc.subcore_barrier()

  return pl.kernel(
      body,
      out_type=jax.ShapeDtypeStruct((NC, n_rows, out_w), jnp.float32),
      mesh=mesh,
      scratch_types=scratch,
      compiler_params=pltpu.CompilerParams(use_tc_tiling_on_sc=False),
  )


def _rows(w):
  return pl.BlockSpec((BR, w), lambda i: (i, 0))


def _prows(w):
  return pl.BlockSpec((NC, BR, w), lambda i: (0, i, 0))


def _full2(a):
  return pl.BlockSpec(a.shape, lambda i: (0, 0))


def _tc_call(body, n, out_ws, in_arrays, in_specs):
  return pl.pallas_call(
      body,
      grid=(n // BR,),
      in_specs=in_specs,
      out_specs=tuple(_rows(w) for w in out_ws),
      out_shape=tuple(
          jax.ShapeDtypeStruct((n, w), jnp.float32) for w in out_ws),
  )(*in_arrays)


def _dot(a, b):
  return jnp.dot(a, b, preferred_element_type=jnp.float32)


def _tc1_body(dp, x, dinv_o, z1_o):
  deg = dp[0, :, :1] + dp[1, :, :1] + 1.0
  dinv = lax.rsqrt(deg)
  dinv_o[...] = dinv
  z1_o[...] = dinv * x[...]


def _tc2_body(sp, z1, dinv, u1_o, z2_o):
  dv = dinv[...]
  u1 = dv * (sp[0] + sp[1] + z1[...])
  u1_o[...] = u1
  z2_o[...] = dv * u1


def _tc3a_body(x, u1, W10, b10, W11, b11, W20a, W21a, W22a,
               qp_o, g1p_o, g2p_o):
  h01 = jnp.concatenate([
      _dot(x[...], W10[...]) + b10[...],
      _dot(u1[...], W11[...]) + b11[...],
  ], axis=1)
  h01 = jnp.maximum(h01, 0.0)
  qp_o[...] = _dot(h01, W20a[...])
  g1p_o[...] = _dot(h01, W21a[...])
  g2p_o[...] = _dot(h01, W22a[...])


def _tc3b_body(sp, z2, dinv, W12, b12, W20b, b20, W21b, W22b,
               qp, g1p, g2p, q0_o, g_o):
  dv = dinv[...]
  u2 = dv * (sp[0] + sp[1] + z2[...])
  h2r = jnp.maximum(_dot(u2, W12[...]) + b12[...], 0.0)
  q0_o[...] = qp[...] + _dot(h2r, W20b[...]) + b20[...]
  zpad = jnp.zeros((h2r.shape[0], 4), jnp.float32)
  g_o[...] = dv * jnp.concatenate([
      g1p[...] + _dot(h2r, W21b[...]), zpad,
      g2p[...] + _dot(h2r, W22b[...]), zpad,
  ], axis=1)


def _tc4_body(sp, g, dinv, vz_o):
  dv = dinv[...]
  v = dv * (sp[0] + sp[1] + g[...])
  vz_o[...] = jnp.concatenate([dv * v[:, 64:], v[:, :64]], axis=1)


def _tc5a_body(q0, vz, b21, fcWa, outp_o):
  h2a = jnp.concatenate([q0[...], vz[:, 64:124] + b21[...]], axis=1)
  h2a = jnp.maximum(h2a, 0.0)
  outp_o[...] = _dot(h2a, fcWa[...])


def _tc5b_body(sp, vz, dinv, b22, fcWb, fcb, outp, out_o):
  dv = dinv[...]
  z4 = vz[:, :64]
  w = dv * (sp[0, :, :64] + sp[1, :, :64] + z4)
  h2b = jnp.maximum(w[:, :60] + b22[...], 0.0)
  out_o[...] = outp[...] + _dot(h2b, fcWb[...]) + fcb[...]


def kernel(x, edge_index, W1_0, W1_1, W1_2, b1_0, b1_1, b1_2,
           W2_0, W2_1, W2_2, b2_0, b2_1, b2_2, fc_W, fc_b):
  n, d_in = x.shape
  e = edge_index.shape[1]
  src = edge_index[0].astype(jnp.int32)
  dst = edge_index[1].astype(jnp.int32)

  ch = None
  for c in range(128, 0, -1):
    if e % (NW * c) == 0:
      ch = c
      break
  if ch is not None:
    pad_rows = 0
    ep = e
  else:
    ch = 64
    ep = -(-e // (NW * ch)) * (NW * ch)
    pad_rows = 16
    fill = jnp.full((ep - e,), n, jnp.int32)
    src = jnp.concatenate([src, fill])
    dst = jnp.concatenate([dst, fill])
  nchunk = ep // (NW * ch)
  src_a = (2 * src).reshape(NW, nchunk, ch)
  src_b = (2 * src + 1).reshape(NW, nchunk, ch)
  dst_r = dst.reshape(NW, nchunk, ch)

  def flat(z):
    if pad_rows:
      z = jnp.concatenate(
          [z, jnp.zeros((pad_rows, z.shape[1]), z.dtype)], axis=0)
    return z.reshape(2 * z.shape[0], 64)

  deg_k = _make_sc_scatter(n, pad_rows, 16, nchunk, ch, 1, deg_mode=True)
  sc2 = _make_sc_scatter(n, pad_rows, 64, nchunk, ch, 2)
  sc1 = _make_sc_scatter(n, pad_rows, 64, nchunk, ch, 1)

  degp = deg_k(dst_r)

  b10 = b1_0.reshape(1, -1)
  b11 = b1_1.reshape(1, -1)
  b12 = b1_2.reshape(1, -1)
  b20 = b2_0.reshape(1, -1)
  b21 = b2_1.reshape(1, -1)
  b22 = b2_2.reshape(1, -1)
  fcb = fc_b.reshape(1, -1)

  dinv, z1 = _tc_call(
      _tc1_body, n, (1, d_in),
      (degp, x), (_prows(128), _rows(d_in)))

  sp = sc2(flat(z1), src_a, src_b, dst_r)
  u1, z2 = _tc_call(
      _tc2_body, n, (d_in, d_in),
      (sp, z1, dinv), (_prows(128), _rows(128), _rows(1)))

  sp = sc2(flat(z2), src_a, src_b, dst_r)
  qp, g1p, g2p = _tc_call(
      _tc3a_body, n, (60, 60, 60),
      (x, u1, W1_0, b10, W1_1, b11, W2_0[:200], W2_1[:200], W2_2[:200]),
      (_rows(d_in), _rows(d_in), _full2(W1_0), _full2(b10),
       _full2(W1_1), _full2(b11),
       pl.BlockSpec((200, 60), lambda i: (0, 0)),
       pl.BlockSpec((200, 60), lambda i: (0, 0)),
       pl.BlockSpec((200, 60), lambda i: (0, 0))))
  q0, g = _tc_call(
      _tc3b_body, n, (60, 128),
      (sp, z2, dinv, W1_2, b12, W2_0[200:], b20, W2_1[200:], W2_2[200:],
       qp, g1p, g2p),
      (_prows(128), _rows(128), _rows(1), _full2(W1_2), _full2(b12),
       pl.BlockSpec((100, 60), lambda i: (0, 0)), _full2(b20),
       pl.BlockSpec((100, 60), lambda i: (0, 0)),
       pl.BlockSpec((100, 60), lambda i: (0, 0)),
       _rows(60), _rows(60), _rows(60)))

  sp = sc2(flat(g), src_a, src_b, dst_r)
  (vz,) = _tc_call(
      _tc4_body, n, (128,),
      (sp, g, dinv), (_prows(128), _rows(128), _rows(1)))

  sp = sc1(flat(vz), src_a, dst_r)
  (outp,) = _tc_call(
      _tc5a_body, n, (128,),
      (q0, vz, b21, fc_W[:120]),
      (_rows(60), _rows(128), _full2(b21),
       pl.BlockSpec((120, 128), lambda i: (0, 0))))
  (out,) = _tc_call(
      _tc5b_body, n, (128,),
      (sp, vz, dinv, b22, fc_W[120:], fcb, outp),
      (_prows(128), _rows(128), _rows(1), _full2(b22),
       pl.BlockSpec((60, 128), lambda i: (0, 0)), _full2(fcb),
       _rows(128)))
  return out


# speedup vs baseline: 35.8102x; 1.0669x over previous
import jax
import jax.numpy as jnp
from jax import lax
from jax.experimental import pallas as pl
from jax.experimental.pallas import tpu as pltpu
from jax.experimental.pallas import tpu_sc as plsc

NC = 2
NS = 16
NW = NC * NS
LANES = 16
BR = 2000


def _make_sc_scatter(n_rows, pad_rows, W, nchunk, ch, npass, deg_mode=False,
                     dt=jnp.float32):
  NZ = 128
  acc_rows = -(-(n_rows + pad_rows) // NZ) * NZ
  nzch = acc_rows // NZ
  wb = next(c for c in range(128, 0, -8) if n_rows % c == 0)
  nwb = n_rows // wb
  out_w = 128

  mesh = plsc.VectorSubcoreMesh(
      core_axis_name="c", subcore_axis_name="s",
      num_cores=NC, num_subcores=NS)

  NB = 2
  if not deg_mode:
    for cand in (5, 4):
      if nchunk % cand == 0 and nchunk >= 2 * cand:
        NB = cand
        break

  LN = 32 if dt == jnp.bfloat16 else 16
  scratch = [
      pltpu.VMEM((nchunk, ch), jnp.int32),
      pltpu.VMEM((nchunk, ch), jnp.int32),
  ] + [pltpu.VMEM((ch, W), dt) for _ in range(NB)] + [
      pltpu.VMEM((NZ, W), dt),
      pltpu.VMEM_SHARED((acc_rows, W), dt),
  ] + [pltpu.SemaphoreType.DMA for _ in range(2 * NB)]

  n_src = 0 if deg_mode else npass

  def body(*refs):
    z_hbm = None if deg_mode else refs[0]
    src_refs = refs[1:1 + n_src] if not deg_mode else ()
    k0 = (1 + n_src) if not deg_mode else 0
    dst_hbm, out_hbm = refs[k0:k0 + 2]
    src_v, dst_v = refs[k0 + 2:k0 + 4]
    bufs = refs[k0 + 4:k0 + 4 + NB]
    zbuf = refs[k0 + 4 + NB]
    acc = refs[k0 + 5 + NB]
    gsems = refs[k0 + 6 + NB:k0 + 6 + 2 * NB]
    ssems = refs[k0 + 6 + 2 * NB:]
    bufa, sema = bufs[0], gsems[0]
    c = lax.axis_index("c")
    s = lax.axis_index("s")
    wid = c * NS + s

    @pl.loop(0, NZ)
    def _(i):
      @pl.loop(0, W // LN)
      def _(k):
        zbuf[i, pl.ds(k * LN, LN)] = jnp.zeros((LN,), dt)

    pltpu.sync_copy(dst_hbm.at[wid], dst_v)
    if deg_mode:
      @pl.loop(0, ch)
      def _(i):
        @pl.loop(0, W // LN)
        def _(k):
          bufa[i, pl.ds(k * LN, LN)] = jnp.ones((LN,), dt)

    for p in range(npass):
      @pl.loop(0, -(-nzch // NS))
      def _(k):
        idx = k * NS + s

        @pl.when(idx < nzch)
        def _():
          pltpu.sync_copy(zbuf, acc.at[pl.ds(idx * NZ, NZ)])

      if not deg_mode:
        pltpu.sync_copy(src_refs[p].at[wid], src_v)

      plsc.subcore_barrier()

      if not deg_mode:
        if NB > 2:
          for k in range(NB):
            pltpu.async_copy(z_hbm.at[src_v.at[k]], bufs[k], gsems[k])

          @pl.loop(0, nchunk // NB)
          def _(j):
            base = NB * j
            for k in range(NB):
              pltpu.make_async_copy(
                  z_hbm.at[src_v.at[base + k]], bufs[k], gsems[k]).wait()
              pltpu.async_copy(
                  bufs[k], acc.at[dst_v.at[base + k]], ssems[k], add=True)
            for k in range(NB):
              pltpu.make_async_copy(
                  bufs[k], acc.at[dst_v.at[base + k]], ssems[k]).wait()

              @pl.when(base + NB + k < nchunk)
              def _(k=k):
                pltpu.async_copy(
                    z_hbm.at[src_v.at[base + NB + k]], bufs[k], gsems[k])
        else:
          bufb, semb = bufs[1], gsems[1]
          pltpu.async_copy(z_hbm.at[src_v.at[0]], bufa, sema)

          @pl.loop(0, nchunk // 2)
          def _(j):
            pltpu.make_async_copy(
                z_hbm.at[src_v.at[2 * j]], bufa, sema).wait()
            pltpu.async_copy(z_hbm.at[src_v.at[2 * j + 1]], bufb, semb)
            pltpu.sync_copy(bufa, acc.at[dst_v.at[2 * j]], add=True)
            pltpu.make_async_copy(
                z_hbm.at[src_v.at[2 * j + 1]], bufb, semb).wait()

            @pl.when(2 * j + 2 < nchunk)
            def _():
              pltpu.async_copy(z_hbm.at[src_v.at[2 * j + 2]], bufa, sema)

            pltpu.sync_copy(bufb, acc.at[dst_v.at[2 * j + 1]], add=True)

          if nchunk % 2 == 1:
            pltpu.make_async_copy(
                z_hbm.at[src_v.at[nchunk - 1]], bufa, sema).wait()
            pltpu.sync_copy(bufa, acc.at[dst_v.at[nchunk - 1]], add=True)
      elif nchunk % 8 == 0:
        @pl.loop(0, nchunk // 8)
        def _(j):
          for k in range(8):
            pltpu.async_copy(bufa, acc.at[dst_v.at[8 * j + k]],
                             ssems[0], add=True)
          for k in range(8):
            pltpu.make_async_copy(bufa, acc.at[dst_v.at[8 * j + k]],
                                  ssems[0]).wait()
      else:
        @pl.loop(0, nchunk)
        def _(j):
          pltpu.sync_copy(bufa, acc.at[dst_v.at[j]], add=True)

      plsc.subcore_barrier()

      @pl.loop(0, -(-nwb // NS))
      def _(k):
        idx = k * NS + s

        @pl.when(idx < nwb)
        def _():
          pltpu.sync_copy(acc.at[pl.ds(idx * wb, wb)],
                          out_hbm.at[c, pl.ds(idx * wb, wb),
                                     pl.ds(p * W, W)])

      if p + 1 < npass:
        plsc.subcore_barrier()

  return pl.kernel(
      body,
      out_type=jax.ShapeDtypeStruct((NC, n_rows, out_w), dt),
      mesh=mesh,
      scratch_types=scratch,
      compiler_params=pltpu.CompilerParams(use_tc_tiling_on_sc=False),
  )


def _rows(w):
  return pl.BlockSpec((BR, w), lambda i: (i, 0))


def _prows(w):
  return pl.BlockSpec((NC, BR, w), lambda i: (0, i, 0))


def _full2(a):
  return pl.BlockSpec(a.shape, lambda i: (0, 0))


def _tc_call(body, n, out_ws, in_arrays, in_specs, out_dts=None):
  if out_dts is None:
    out_dts = (jnp.float32,) * len(out_ws)
  return pl.pallas_call(
      body,
      grid=(n // BR,),
      in_specs=in_specs,
      out_specs=tuple(_rows(w) for w in out_ws),
      out_shape=tuple(
          jax.ShapeDtypeStruct((n, w), d) for w, d in zip(out_ws, out_dts)),
  )(*in_arrays)


def _f32(ref_val):
  return ref_val.astype(jnp.float32)


def _dot(a, b):
  return jnp.dot(a, b, preferred_element_type=jnp.float32)


def _tc1_body(dp, x, dinv_o, z1_o):
  deg = dp[0, :, :1] + dp[1, :, :1] + 1.0
  dinv = lax.rsqrt(deg)
  dinv_o[...] = dinv
  z1_o[...] = (dinv * x[...]).astype(z1_o.dtype)


def _tc2_body(sp, z1, dinv, u1_o, z2_o):
  dv = dinv[...]
  u1 = dv * (_f32(sp[0]) + _f32(sp[1]) + _f32(z1[...]))
  u1_o[...] = u1
  z2_o[...] = (dv * u1).astype(z2_o.dtype)


def _tc3a_body(x, u1, W10, b10, W11, b11, W20a, W21a, W22a,
               qp_o, g1p_o, g2p_o):
  h01 = jnp.concatenate([
      _dot(x[...], W10[...]) + b10[...],
      _dot(u1[...], W11[...]) + b11[...],
  ], axis=1)
  h01 = jnp.maximum(h01, 0.0)
  qp_o[...] = _dot(h01, W20a[...])
  g1p_o[...] = _dot(h01, W21a[...])
  g2p_o[...] = _dot(h01, W22a[...])


def _tc3b_body(sp, z2, dinv, W12, b12, W20b, b20, W21b, W22b,
               qp, g1p, g2p, q0_o, g_o):
  dv = dinv[...]
  u2 = dv * (_f32(sp[0]) + _f32(sp[1]) + _f32(z2[...]))
  h2r = jnp.maximum(_dot(u2, W12[...]) + b12[...], 0.0)
  q0_o[...] = qp[...] + _dot(h2r, W20b[...]) + b20[...]
  zpad = jnp.zeros((h2r.shape[0], 4), jnp.float32)
  g_o[...] = (dv * jnp.concatenate([
      g1p[...] + _dot(h2r, W21b[...]), zpad,
      g2p[...] + _dot(h2r, W22b[...]), zpad,
  ], axis=1)).astype(g_o.dtype)


def _tc4_body(sp, g, dinv, vz_o):
  dv = dinv[...]
  v = dv * (_f32(sp[0]) + _f32(sp[1]) + _f32(g[...]))
  vz_o[...] = jnp.concatenate(
      [dv * v[:, 64:], v[:, :64]], axis=1).astype(vz_o.dtype)


def _tc5a_body(q0, vz, b21, fcWa, outp_o):
  h2a = jnp.concatenate([q0[...], _f32(vz[:, 64:124]) + b21[...]], axis=1)
  h2a = jnp.maximum(h2a, 0.0)
  outp_o[...] = _dot(h2a, fcWa[...])


def _tc5b_body(sp, vz, dinv, b22, fcWb, fcb, outp, out_o):
  dv = dinv[...]
  z4 = _f32(vz[:, :64])
  w = dv * (_f32(sp[0, :, :64]) + _f32(sp[1, :, :64]) + z4)
  h2b = jnp.maximum(w[:, :60] + b22[...], 0.0)
  out_o[...] = outp[...] + _dot(h2b, fcWb[...]) + fcb[...]


def kernel(x, edge_index, W1_0, W1_1, W1_2, b1_0, b1_1, b1_2,
           W2_0, W2_1, W2_2, b2_0, b2_1, b2_2, fc_W, fc_b):
  n, d_in = x.shape
  e = edge_index.shape[1]
  src = edge_index[0].astype(jnp.int32)
  dst = edge_index[1].astype(jnp.int32)

  ch = None
  for c in range(128, 0, -1):
    if e % (NW * c) == 0:
      ch = c
      break
  if ch is not None:
    pad_rows = 0
    ep = e
  else:
    ch = 64
    ep = -(-e // (NW * ch)) * (NW * ch)
    pad_rows = 16
    fill = jnp.full((ep - e,), n, jnp.int32)
    src = jnp.concatenate([src, fill])
    dst = jnp.concatenate([dst, fill])
  nchunk = ep // (NW * ch)
  src_a = (2 * src).reshape(NW, nchunk, ch)
  src_b = (2 * src + 1).reshape(NW, nchunk, ch)
  dst_r = dst.reshape(NW, nchunk, ch)

  def flat(z):
    if pad_rows:
      z = jnp.concatenate(
          [z, jnp.zeros((pad_rows, z.shape[1]), z.dtype)], axis=0)
    return z.reshape(2 * z.shape[0], 64)

  deg_k = _make_sc_scatter(n, pad_rows, 16, nchunk, ch, 1, deg_mode=True)
  sc2 = _make_sc_scatter(n, pad_rows, 64, nchunk, ch, 2, dt=jnp.bfloat16)
  sc1 = _make_sc_scatter(n, pad_rows, 64, nchunk, ch, 1, dt=jnp.bfloat16)

  degp = deg_k(dst_r)

  b10 = b1_0.reshape(1, -1)
  b11 = b1_1.reshape(1, -1)
  b12 = b1_2.reshape(1, -1)
  b20 = b2_0.reshape(1, -1)
  b21 = b2_1.reshape(1, -1)
  b22 = b2_2.reshape(1, -1)
  fcb = fc_b.reshape(1, -1)

  dinv, z1 = _tc_call(
      _tc1_body, n, (1, d_in),
      (degp, x), (_prows(128), _rows(d_in)),
      out_dts=(jnp.float32, jnp.bfloat16))

  sp = sc2(flat(z1), src_a, src_b, dst_r)
  u1, z2 = _tc_call(
      _tc2_body, n, (d_in, d_in),
      (sp, z1, dinv), (_prows(128), _rows(128), _rows(1)),
      out_dts=(jnp.float32, jnp.bfloat16))

  sp = sc2(flat(z2), src_a, src_b, dst_r)
  qp, g1p, g2p = _tc_call(
      _tc3a_body, n, (60, 60, 60),
      (x, u1, W1_0, b10, W1_1, b11, W2_0[:200], W2_1[:200], W2_2[:200]),
      (_rows(d_in), _rows(d_in), _full2(W1_0), _full2(b10),
       _full2(W1_1), _full2(b11),
       pl.BlockSpec((200, 60), lambda i: (0, 0)),
       pl.BlockSpec((200, 60), lambda i: (0, 0)),
       pl.BlockSpec((200, 60), lambda i: (0, 0))))
  q0, g = _tc_call(
      _tc3b_body, n, (60, 128),
      (sp, z2, dinv, W1_2, b12, W2_0[200:], b20, W2_1[200:], W2_2[200:],
       qp, g1p, g2p),
      (_prows(128), _rows(128), _rows(1), _full2(W1_2), _full2(b12),
       pl.BlockSpec((100, 60), lambda i: (0, 0)), _full2(b20),
       pl.BlockSpec((100, 60), lambda i: (0, 0)),
       pl.BlockSpec((100, 60), lambda i: (0, 0)),
       _rows(60), _rows(60), _rows(60)),
      out_dts=(jnp.float32, jnp.bfloat16))

  sp = sc2(flat(g), src_a, src_b, dst_r)
  (vz,) = _tc_call(
      _tc4_body, n, (128,),
      (sp, g, dinv), (_prows(128), _rows(128), _rows(1)),
      out_dts=(jnp.bfloat16,))

  sp = sc1(flat(vz), src_a, dst_r)
  (outp,) = _tc_call(
      _tc5a_body, n, (128,),
      (q0, vz, b21, fc_W[:120]),
      (_rows(60), _rows(128), _full2(b21),
       pl.BlockSpec((120, 128), lambda i: (0, 0))))
  (out,) = _tc_call(
      _tc5b_body, n, (128,),
      (sp, vz, dinv, b22, fc_W[120:], fcb, outp),
      (_prows(128), _rows(128), _rows(1), _full2(b22),
       pl.BlockSpec((60, 128), lambda i: (0, 0)), _full2(fcb),
       _rows(128)))
  return out
